# Initial kernel scaffold; baseline (speedup 1.0000x reference)
import jax
import jax.numpy as jnp
from jax.experimental import pallas as pl


def kernel(x, edge_index, W1, b1, W2, b2):
    raise NotImplementedError("write your pallas kernel here")


# speedup vs baseline: 15.7091x; 15.7091x over previous
import functools

import jax
import jax.numpy as jnp
from jax import lax
from jax.experimental import pallas as pl
from jax.experimental.pallas import tpu as pltpu
from jax.experimental.pallas import tpu_sc as plsc

NC = 2
NS = 16
NW = NC * NS
CH = 128


def _mesh():
    return plsc.VectorSubcoreMesh(core_axis_name="c", subcore_axis_name="s")


def _sc_degree(dst, n_pad):
    e = dst.shape[0]
    per_w = e // NW
    full = per_w // CH
    rem = per_w - full * CH
    per_tile = n_pad // NS

    def body(dst_hbm, out_hbm, didx, didx_r, ones_v, zb, acc):
        c = lax.axis_index("c")
        s = lax.axis_index("s")
        wid = s * NC + c

        def fill_ones(i, carry):
            ones_v[pl.ds(i * 16, 16)] = jnp.ones((16,), jnp.float32)
            return carry

        lax.fori_loop(0, CH // 16, fill_ones, 0)

        def fill_z(i, carry):
            zb[pl.ds(i * 16, 16)] = jnp.zeros((16,), jnp.float32)
            return carry

        lax.fori_loop(0, per_tile // 16, fill_z, 0)
        pltpu.sync_copy(zb, acc.at[pl.ds(s * per_tile, per_tile)])
        plsc.subcore_barrier()

        base0 = wid * per_w

        def chunk(i, carry):
            pltpu.sync_copy(dst_hbm.at[pl.ds(base0 + i * CH, CH)], didx)
            pltpu.sync_copy(ones_v, acc.at[didx], add=True)
            return carry

        lax.fori_loop(0, full, chunk, 0)
        if rem:
            pltpu.sync_copy(dst_hbm.at[pl.ds(base0 + full * CH, rem)], didx_r)
            pltpu.sync_copy(ones_v.at[pl.ds(0, rem)], acc.at[didx_r], add=True)
        plsc.subcore_barrier()
        pltpu.sync_copy(acc.at[pl.ds(s * per_tile, per_tile)],
                        out_hbm.at[pl.ds(c * n_pad + s * per_tile, per_tile)])

    return pl.kernel(
        body,
        out_type=jax.ShapeDtypeStruct((NC * n_pad,), jnp.float32),
        mesh=_mesh(),
        scratch_types=[
            pltpu.VMEM((CH,), jnp.int32),
            pltpu.VMEM((max(rem, 16),), jnp.int32),
            pltpu.VMEM((CH,), jnp.float32),
            pltpu.VMEM((per_tile,), jnp.float32),
            pltpu.VMEM_SHARED((n_pad,), jnp.float32),
        ],
    )(dst)


def _sc_aggregate(h, zeros_h, src, dst):
    n, d = h.shape
    e = src.shape[0]
    per_w = e // NW
    full = per_w // CH
    rem = per_w - full * CH
    ir = 1000
    ni = n // ir

    def body(h_hbm, z_hbm, src_hbm, dst_hbm, out_hbm,
             sidx, didx, sidx_r, didx_r, rows, rows_r, acc, sem):
        c = lax.axis_index("c")
        s = lax.axis_index("s")
        wid = s * NC + c
        rslice = pl.ds(s * ir, ir)

        @pl.when(jnp.logical_and(s < ni, c == 0))
        def _():
            pltpu.sync_copy(h_hbm.at[rslice], acc.at[rslice])

        @pl.when(jnp.logical_and(s < ni, c != 0))
        def _():
            pltpu.sync_copy(z_hbm.at[rslice], acc.at[rslice])

        plsc.subcore_barrier()

        base0 = wid * per_w

        def chunk(i, carry):
            b = base0 + i * CH
            pltpu.sync_copy(src_hbm.at[pl.ds(b, CH)], sidx)
            pltpu.sync_copy(dst_hbm.at[pl.ds(b, CH)], didx)
            pltpu.async_copy(h_hbm.at[sidx], rows, sem).wait()
            pltpu.sync_copy(rows, acc.at[didx], add=True)
            return carry

        lax.fori_loop(0, full, chunk, 0)
        if rem:
            b = base0 + full * CH
            pltpu.sync_copy(src_hbm.at[pl.ds(b, rem)], sidx_r)
            pltpu.sync_copy(dst_hbm.at[pl.ds(b, rem)], didx_r)
            pltpu.async_copy(h_hbm.at[sidx_r], rows_r, sem).wait()
            pltpu.sync_copy(rows_r, acc.at[didx_r], add=True)
        plsc.subcore_barrier()

        @pl.when(s < ni)
        def _():
            pltpu.sync_copy(acc.at[rslice],
                            out_hbm.at[pl.ds(c * n + s * ir, ir)])

    return pl.kernel(
        body,
        out_type=jax.ShapeDtypeStruct((NC * n, d), jnp.float32),
        mesh=_mesh(),
        scratch_types=[
            pltpu.VMEM((CH,), jnp.int32),
            pltpu.VMEM((CH,), jnp.int32),
            pltpu.VMEM((max(rem, 16),), jnp.int32),
            pltpu.VMEM((max(rem, 16),), jnp.int32),
            pltpu.VMEM((CH, d), jnp.float32),
            pltpu.VMEM((max(rem, 16), d), jnp.float32),
            pltpu.VMEM_SHARED((n, d), jnp.float32),
            pltpu.SemaphoreType.DMA,
        ],
    )(h, zeros_h, src, dst)


def _tc_layer1(x, w1, deg0, deg1, bs):
    n, d_in = x.shape
    d_hid = w1.shape[0]

    def body(x_ref, w_ref, d0_ref, d1_ref, o_ref):
        dv = lax.rsqrt(d0_ref[...] + d1_ref[...] + 1.0)
        hm = lax.dot_general(x_ref[...], w_ref[...], (((1,), (1,)), ((), ())),
                             preferred_element_type=jnp.float32)
        o_ref[...] = hm * dv

    return pl.pallas_call(
        body,
        grid=(n // bs,),
        in_specs=[
            pl.BlockSpec((bs, d_in), lambda i: (i, 0)),
            pl.BlockSpec((d_hid, d_in), lambda i: (0, 0)),
            pl.BlockSpec((bs, 1), lambda i: (i, 0)),
            pl.BlockSpec((bs, 1), lambda i: (i, 0)),
        ],
        out_specs=pl.BlockSpec((bs, d_hid), lambda i: (i, 0)),
        out_shape=jax.ShapeDtypeStruct((n, d_hid), jnp.float32),
    )(x, w1, deg0, deg1)


def _tc_layer2(p0, p1, deg0, deg1, b1, w2, bs):
    n, d_hid = p0.shape
    d_out = w2.shape[0]

    def body(p0_ref, p1_ref, d0_ref, d1_ref, b_ref, w_ref, o_ref):
        dv = lax.rsqrt(d0_ref[...] + d1_ref[...] + 1.0)
        x1 = jnp.maximum((p0_ref[...] + p1_ref[...]) * dv + b_ref[...], 0.0)
        h2 = lax.dot_general(x1, w_ref[...], (((1,), (1,)), ((), ())),
                             preferred_element_type=jnp.float32)
        o_ref[...] = h2 * dv

    return pl.pallas_call(
        body,
        grid=(n // bs,),
        in_specs=[
            pl.BlockSpec((bs, d_hid), lambda i: (i, 0)),
            pl.BlockSpec((bs, d_hid), lambda i: (i, 0)),
            pl.BlockSpec((bs, 1), lambda i: (i, 0)),
            pl.BlockSpec((bs, 1), lambda i: (i, 0)),
            pl.BlockSpec((1, d_hid), lambda i: (0, 0)),
            pl.BlockSpec((d_out, d_hid), lambda i: (0, 0)),
        ],
        out_specs=pl.BlockSpec((bs, d_out), lambda i: (i, 0)),
        out_shape=jax.ShapeDtypeStruct((n, d_out), jnp.float32),
    )(p0, p1, deg0, deg1, b1, w2)


def _tc_layer3(q0, q1, deg0, deg1, b2, bs, d_real):
    n, d_pad = q0.shape

    def body(q0_ref, q1_ref, d0_ref, d1_ref, b_ref, o_ref):
        dv = lax.rsqrt(d0_ref[...] + d1_ref[...] + 1.0)
        z = ((q0_ref[...] + q1_ref[...]) * dv + b_ref[...])[:, :d_real]
        m = jnp.max(z, axis=1, keepdims=True)
        ez = jnp.exp(z - m)
        se = jnp.sum(ez, axis=1, keepdims=True)
        o_ref[...] = z - m - jnp.log(se)

    return pl.pallas_call(
        body,
        grid=(n // bs,),
        in_specs=[
            pl.BlockSpec((bs, d_pad), lambda i: (i, 0)),
            pl.BlockSpec((bs, d_pad), lambda i: (i, 0)),
            pl.BlockSpec((bs, 1), lambda i: (i, 0)),
            pl.BlockSpec((bs, 1), lambda i: (i, 0)),
            pl.BlockSpec((1, d_pad), lambda i: (0, 0)),
        ],
        out_specs=pl.BlockSpec((bs, d_real), lambda i: (i, 0)),
        out_shape=jax.ShapeDtypeStruct((n, d_real), jnp.float32),
    )(q0, q1, deg0, deg1, b2)


def kernel(x, edge_index, W1, b1, W2, b2):
    n, _ = x.shape
    d_hid = W1.shape[0]
    d_out = W2.shape[0]
    src = edge_index[0]
    dst = edge_index[1]

    per_tile = -(-n // NS)
    per_tile += (-per_tile) % 16
    n_pad = per_tile * NS

    degp = _sc_degree(dst, n_pad)
    deg0 = degp[:n].reshape(n, 1)
    deg1 = degp[n_pad:n_pad + n].reshape(n, 1)

    bs = 1000
    zeros_nd = jnp.zeros((n, d_hid), jnp.float32)
    w2p = jnp.concatenate(
        [W2, jnp.zeros((d_hid - d_out, W2.shape[1]), jnp.float32)], axis=0)
    b2p = jnp.concatenate(
        [b2, jnp.zeros((d_hid - d_out,), jnp.float32)]).reshape(1, d_hid)

    h1 = _tc_layer1(x, W1, deg0, deg1, bs)
    p = _sc_aggregate(h1, zeros_nd, src, dst)
    h2 = _tc_layer2(p[:n], p[n:], deg0, deg1, b1.reshape(1, d_hid), w2p, bs)
    q = _sc_aggregate(h2, zeros_nd, src, dst)
    return _tc_layer3(q[:n], q[n:], deg0, deg1, b2p, bs, d_out)


# speedup vs baseline: 29.5045x; 1.8782x over previous
import functools

import jax
import jax.numpy as jnp
from jax import lax
from jax.experimental import pallas as pl
from jax.experimental.pallas import tpu as pltpu
from jax.experimental.pallas import tpu_sc as plsc

NC = 2
NS = 16
NW = NC * NS
CH = 128


def _mesh():
    return plsc.VectorSubcoreMesh(core_axis_name="c", subcore_axis_name="s")


def _sc_degree(dst2d, n_pad):
    chunks = dst2d.shape[0] // NW
    per_tile = n_pad // NS

    def body(dst_hbm, out_hbm, didx, ones_v, zb, acc):
        c = lax.axis_index("c")
        s = lax.axis_index("s")
        wid = s * NC + c

        def fill_ones(i, carry):
            ones_v[pl.ds(i * 16, 16)] = jnp.ones((16,), jnp.float32)
            return carry

        lax.fori_loop(0, CH // 16, fill_ones, 0)

        def fill_z(i, carry):
            zb[pl.ds(i * 16, 16)] = jnp.zeros((16,), jnp.float32)
            return carry

        lax.fori_loop(0, per_tile // 16, fill_z, 0)
        pltpu.sync_copy(zb, acc.at[pl.ds(s * per_tile, per_tile)])
        pltpu.sync_copy(dst_hbm.at[pl.ds(wid * chunks, chunks)], didx)
        plsc.subcore_barrier()

        def chunk(i, carry):
            pltpu.sync_copy(ones_v, acc.at[didx.at[i]], add=True)
            return carry

        lax.fori_loop(0, chunks, chunk, 0)
        plsc.subcore_barrier()
        pltpu.sync_copy(acc.at[pl.ds(s * per_tile, per_tile)],
                        out_hbm.at[pl.ds(c * n_pad + s * per_tile, per_tile)])

    return pl.kernel(
        body,
        out_type=jax.ShapeDtypeStruct((NC * n_pad,), jnp.float32),
        mesh=_mesh(),
        scratch_types=[
            pltpu.VMEM((chunks, CH), jnp.int32),
            pltpu.VMEM((CH,), jnp.float32),
            pltpu.VMEM((per_tile,), jnp.float32),
            pltpu.VMEM_SHARED((n_pad,), jnp.float32),
        ],
    )(dst2d)


def _sc_aggregate(h, zeros_h, src2d, dst2d):
    n, d = h.shape
    chunks = src2d.shape[0] // NW
    ir = 1000
    ni = n // ir

    def body(h_hbm, z_hbm, src_hbm, dst_hbm, out_hbm,
             sidx, didx, rows, acc, gs0, gs1):
        c = lax.axis_index("c")
        s = lax.axis_index("s")
        wid = s * NC + c
        rslice = pl.ds(s * ir, ir)

        @pl.when(jnp.logical_and(s < ni, c == 0))
        def _():
            pltpu.sync_copy(h_hbm.at[rslice], acc.at[rslice])

        @pl.when(jnp.logical_and(s < ni, c != 0))
        def _():
            pltpu.sync_copy(z_hbm.at[rslice], acc.at[rslice])

        @pl.when(s == ni)
        def _():
            pltpu.sync_copy(z_hbm.at[pl.ds(0, CH)], acc.at[pl.ds(n, CH)])

        plsc.subcore_barrier()

        hc = chunks // 2
        for half in range(2):
            off = wid * chunks + half * hc
            pltpu.sync_copy(src_hbm.at[pl.ds(off, hc)], sidx)
            pltpu.sync_copy(dst_hbm.at[pl.ds(off, hc)], didx)
            pltpu.async_copy(h_hbm.at[sidx.at[0]], rows.at[0], gs0)
            pltpu.async_copy(h_hbm.at[sidx.at[1]], rows.at[1], gs1)

            def pair(g, carry):
                i0 = 2 * g
                pltpu.make_async_copy(h_hbm.at[sidx.at[i0]], rows.at[0],
                                      gs0).wait()
                pltpu.sync_copy(rows.at[0], acc.at[didx.at[i0]], add=True)

                @pl.when(i0 + 2 < hc)
                def _():
                    pltpu.async_copy(h_hbm.at[sidx.at[i0 + 2]], rows.at[0],
                                     gs0)

                pltpu.make_async_copy(h_hbm.at[sidx.at[i0 + 1]], rows.at[1],
                                      gs1).wait()
                pltpu.sync_copy(rows.at[1], acc.at[didx.at[i0 + 1]], add=True)

                @pl.when(i0 + 3 < hc)
                def _():
                    pltpu.async_copy(h_hbm.at[sidx.at[i0 + 3]], rows.at[1],
                                     gs1)

                return carry

            lax.fori_loop(0, hc // 2, pair, 0)
        plsc.subcore_barrier()

        @pl.when(s < ni)
        def _():
            pltpu.sync_copy(acc.at[rslice],
                            out_hbm.at[pl.ds(c * n + s * ir, ir)])

    return pl.kernel(
        body,
        out_type=jax.ShapeDtypeStruct((NC * n, d), jnp.float32),
        mesh=_mesh(),
        scratch_types=[
            pltpu.VMEM((chunks // 2, CH), jnp.int32),
            pltpu.VMEM((chunks // 2, CH), jnp.int32),
            pltpu.VMEM((2, CH, d), jnp.float32),
            pltpu.VMEM_SHARED((n + CH, d), jnp.float32),
            pltpu.SemaphoreType.DMA,
            pltpu.SemaphoreType.DMA,
        ],
    )(h, zeros_h, src2d, dst2d)


def _tc_layer1(x, w1, deg0, deg1, bs):
    n, d_in = x.shape
    d_hid = w1.shape[0]

    def body(x_ref, w_ref, d0_ref, d1_ref, o_ref):
        dv = lax.rsqrt(d0_ref[...] + d1_ref[...] + 1.0)
        hm = lax.dot_general(x_ref[...], w_ref[...], (((1,), (1,)), ((), ())),
                             preferred_element_type=jnp.float32)
        o_ref[...] = hm * dv

    return pl.pallas_call(
        body,
        grid=(n // bs,),
        in_specs=[
            pl.BlockSpec((bs, d_in), lambda i: (i, 0)),
            pl.BlockSpec((d_hid, d_in), lambda i: (0, 0)),
            pl.BlockSpec((bs, 1), lambda i: (i, 0)),
            pl.BlockSpec((bs, 1), lambda i: (i, 0)),
        ],
        out_specs=pl.BlockSpec((bs, d_hid), lambda i: (i, 0)),
        out_shape=jax.ShapeDtypeStruct((n, d_hid), jnp.float32),
    )(x, w1, deg0, deg1)


def _tc_layer2(p0, p1, deg0, deg1, b1, w2, bs):
    n, d_hid = p0.shape
    d_out = w2.shape[0]

    def body(p0_ref, p1_ref, d0_ref, d1_ref, b_ref, w_ref, o_ref):
        dv = lax.rsqrt(d0_ref[...] + d1_ref[...] + 1.0)
        x1 = jnp.maximum((p0_ref[...] + p1_ref[...]) * dv + b_ref[...], 0.0)
        h2 = lax.dot_general(x1, w_ref[...], (((1,), (1,)), ((), ())),
                             preferred_element_type=jnp.float32)
        o_ref[...] = h2 * dv

    return pl.pallas_call(
        body,
        grid=(n // bs,),
        in_specs=[
            pl.BlockSpec((bs, d_hid), lambda i: (i, 0)),
            pl.BlockSpec((bs, d_hid), lambda i: (i, 0)),
            pl.BlockSpec((bs, 1), lambda i: (i, 0)),
            pl.BlockSpec((bs, 1), lambda i: (i, 0)),
            pl.BlockSpec((1, d_hid), lambda i: (0, 0)),
            pl.BlockSpec((d_out, d_hid), lambda i: (0, 0)),
        ],
        out_specs=pl.BlockSpec((bs, d_out), lambda i: (i, 0)),
        out_shape=jax.ShapeDtypeStruct((n, d_out), jnp.float32),
    )(p0, p1, deg0, deg1, b1, w2)


def _tc_layer3(q0, q1, deg0, deg1, b2, bs, d_real):
    n, d_pad = q0.shape

    def body(q0_ref, q1_ref, d0_ref, d1_ref, b_ref, o_ref):
        dv = lax.rsqrt(d0_ref[...] + d1_ref[...] + 1.0)
        z = ((q0_ref[...] + q1_ref[...]) * dv + b_ref[...])[:, :d_real]
        m = jnp.max(z, axis=1, keepdims=True)
        ez = jnp.exp(z - m)
        se = jnp.sum(ez, axis=1, keepdims=True)
        o_ref[...] = z - m - jnp.log(se)

    return pl.pallas_call(
        body,
        grid=(n // bs,),
        in_specs=[
            pl.BlockSpec((bs, d_pad), lambda i: (i, 0)),
            pl.BlockSpec((bs, d_pad), lambda i: (i, 0)),
            pl.BlockSpec((bs, 1), lambda i: (i, 0)),
            pl.BlockSpec((bs, 1), lambda i: (i, 0)),
            pl.BlockSpec((1, d_pad), lambda i: (0, 0)),
        ],
        out_specs=pl.BlockSpec((bs, d_real), lambda i: (i, 0)),
        out_shape=jax.ShapeDtypeStruct((n, d_real), jnp.float32),
    )(q0, q1, deg0, deg1, b2)


def kernel(x, edge_index, W1, b1, W2, b2):
    n, _ = x.shape
    d_hid = W1.shape[0]
    d_out = W2.shape[0]
    src = edge_index[0]
    dst = edge_index[1]
    e = src.shape[0]

    per_tile = -(-n // NS)
    per_tile += (-per_tile) % 16
    n_pad = per_tile * NS

    chunks = -(-e // (NW * CH))
    chunks += (-chunks) % 8
    e_pad = NW * CH * chunks
    pad = e_pad - e
    pad_ar = jnp.arange(pad, dtype=jnp.int32)
    src2d = jnp.concatenate([src, pad_ar % n]).reshape(NW * chunks, CH)
    dst2d = jnp.concatenate([dst, n + pad_ar % CH]).reshape(NW * chunks, CH)

    degp = _sc_degree(dst2d, n_pad)
    deg0 = degp[:n].reshape(n, 1)
    deg1 = degp[n_pad:n_pad + n].reshape(n, 1)

    bs = 1000
    zeros_nd = jnp.zeros((n, d_hid), jnp.float32)
    w2p = jnp.concatenate(
        [W2, jnp.zeros((d_hid - d_out, W2.shape[1]), jnp.float32)], axis=0)
    b2p = jnp.concatenate(
        [b2, jnp.zeros((d_hid - d_out,), jnp.float32)]).reshape(1, d_hid)

    h1 = _tc_layer1(x, W1, deg0, deg1, bs)
    p = _sc_aggregate(h1, zeros_nd, src2d, dst2d)
    h2 = _tc_layer2(p[:n], p[n:], deg0, deg1, b1.reshape(1, d_hid), w2p, bs)
    q = _sc_aggregate(h2, zeros_nd, src2d, dst2d)
    return _tc_layer3(q[:n], q[n:], deg0, deg1, b2p, bs, d_out)


# speedup vs baseline: 32.2856x; 1.0943x over previous
import functools

import jax
import jax.numpy as jnp
from jax import lax
from jax.experimental import pallas as pl
from jax.experimental.pallas import tpu as pltpu
from jax.experimental.pallas import tpu_sc as plsc

NC = 2
NS = 16
NW = NC * NS
CH = 128


def _mesh():
    return plsc.VectorSubcoreMesh(core_axis_name="c", subcore_axis_name="s")


def _sc_degree(dst2d, n_pad):
    chunks = dst2d.shape[0] // NW
    per_tile = n_pad // NS

    def body(dst_hbm, out_hbm, didx, ones_v, zb, acc):
        c = lax.axis_index("c")
        s = lax.axis_index("s")
        wid = s * NC + c

        def fill_ones(i, carry):
            ones_v[pl.ds(i * 16, 16)] = jnp.ones((16,), jnp.float32)
            return carry

        lax.fori_loop(0, CH // 16, fill_ones, 0)

        def fill_z(i, carry):
            zb[pl.ds(i * 16, 16)] = jnp.zeros((16,), jnp.float32)
            return carry

        lax.fori_loop(0, per_tile // 16, fill_z, 0)
        pltpu.sync_copy(zb, acc.at[pl.ds(s * per_tile, per_tile)])
        pltpu.sync_copy(dst_hbm.at[pl.ds(wid * chunks, chunks)], didx)
        plsc.subcore_barrier()

        def chunk(i, carry):
            pltpu.sync_copy(ones_v, acc.at[didx.at[i]], add=True)
            return carry

        lax.fori_loop(0, chunks, chunk, 0)
        plsc.subcore_barrier()
        pltpu.sync_copy(acc.at[pl.ds(s * per_tile, per_tile)],
                        out_hbm.at[pl.ds(c * n_pad + s * per_tile, per_tile)])

    return pl.kernel(
        body,
        out_type=jax.ShapeDtypeStruct((NC * n_pad,), jnp.float32),
        mesh=_mesh(),
        scratch_types=[
            pltpu.VMEM((chunks, CH), jnp.int32),
            pltpu.VMEM((CH,), jnp.float32),
            pltpu.VMEM((per_tile,), jnp.float32),
            pltpu.VMEM_SHARED((n_pad,), jnp.float32),
        ],
    )(dst2d)


def _sc_aggregate(h, zeros_h, src2d, dst2d, tc_tiling=True):
    n, d = h.shape
    chunks = src2d.shape[0] // NW
    ir = 1000
    ni = n // ir
    nstage = 1 if d <= 64 else 2
    hc = chunks // nstage

    def body(h_hbm, z_hbm, src_hbm, dst_hbm, out_hbm,
             sidx, didx, rows, acc, gs0, gs1):
        c = lax.axis_index("c")
        s = lax.axis_index("s")
        wid = s * NC + c
        rslice = pl.ds(s * ir, ir)

        @pl.when(jnp.logical_and(s < ni, c == 0))
        def _():
            pltpu.sync_copy(h_hbm.at[rslice], acc.at[rslice])

        @pl.when(jnp.logical_and(s < ni, c != 0))
        def _():
            pltpu.sync_copy(z_hbm.at[rslice], acc.at[rslice])

        @pl.when(s == ni)
        def _():
            pltpu.sync_copy(z_hbm.at[pl.ds(0, CH)], acc.at[pl.ds(n, CH)])

        plsc.subcore_barrier()

        for half in range(nstage):
            off = wid * chunks + half * hc
            pltpu.sync_copy(src_hbm.at[pl.ds(off, hc)], sidx)
            pltpu.sync_copy(dst_hbm.at[pl.ds(off, hc)], didx)
            pltpu.async_copy(h_hbm.at[sidx.at[0]], rows.at[0], gs0)
            pltpu.async_copy(h_hbm.at[sidx.at[1]], rows.at[1], gs1)

            def pair(g, carry):
                i0 = 2 * g
                pltpu.make_async_copy(h_hbm.at[sidx.at[i0]], rows.at[0],
                                      gs0).wait()
                pltpu.sync_copy(rows.at[0], acc.at[didx.at[i0]], add=True)

                @pl.when(i0 + 2 < hc)
                def _():
                    pltpu.async_copy(h_hbm.at[sidx.at[i0 + 2]], rows.at[0],
                                     gs0)

                pltpu.make_async_copy(h_hbm.at[sidx.at[i0 + 1]], rows.at[1],
                                      gs1).wait()
                pltpu.sync_copy(rows.at[1], acc.at[didx.at[i0 + 1]], add=True)

                @pl.when(i0 + 3 < hc)
                def _():
                    pltpu.async_copy(h_hbm.at[sidx.at[i0 + 3]], rows.at[1],
                                     gs1)

                return carry

            lax.fori_loop(0, hc // 2, pair, 0)
        plsc.subcore_barrier()

        @pl.when(s < ni)
        def _():
            pltpu.sync_copy(acc.at[rslice],
                            out_hbm.at[pl.ds(c * n + s * ir, ir)])

    return pl.kernel(
        body,
        out_type=jax.ShapeDtypeStruct((NC * n, d), jnp.float32),
        mesh=_mesh(),
        scratch_types=[
            pltpu.VMEM((hc, CH), jnp.int32),
            pltpu.VMEM((hc, CH), jnp.int32),
            pltpu.VMEM((2, CH, d), jnp.float32),
            pltpu.VMEM_SHARED((n + CH, d), jnp.float32),
            pltpu.SemaphoreType.DMA,
            pltpu.SemaphoreType.DMA,
        ],
        compiler_params=pltpu.CompilerParams(use_tc_tiling_on_sc=tc_tiling),
    )(h, zeros_h, src2d, dst2d)


def _tc_layer1(x, w1, deg0, deg1, bs):
    n, d_in = x.shape
    d_hid = w1.shape[0]

    def body(x_ref, w_ref, d0_ref, d1_ref, o_ref):
        dv = lax.rsqrt(d0_ref[...] + d1_ref[...] + 1.0)
        hm = lax.dot_general(x_ref[...], w_ref[...], (((1,), (1,)), ((), ())),
                             preferred_element_type=jnp.float32)
        o_ref[...] = hm * dv

    return pl.pallas_call(
        body,
        grid=(n // bs,),
        in_specs=[
            pl.BlockSpec((bs, d_in), lambda i: (i, 0)),
            pl.BlockSpec((d_hid, d_in), lambda i: (0, 0)),
            pl.BlockSpec((bs, 1), lambda i: (i, 0)),
            pl.BlockSpec((bs, 1), lambda i: (i, 0)),
        ],
        out_specs=pl.BlockSpec((bs, d_hid), lambda i: (i, 0)),
        out_shape=jax.ShapeDtypeStruct((n, d_hid), jnp.float32),
    )(x, w1, deg0, deg1)


def _tc_layer2(p0, p1, deg0, deg1, b1, w2, bs):
    n, d_hid = p0.shape
    d_out = w2.shape[0]

    def body(p0_ref, p1_ref, d0_ref, d1_ref, b_ref, w_ref, o_ref):
        dv = lax.rsqrt(d0_ref[...] + d1_ref[...] + 1.0)
        x1 = jnp.maximum((p0_ref[...] + p1_ref[...]) * dv + b_ref[...], 0.0)
        h2 = lax.dot_general(x1, w_ref[...], (((1,), (1,)), ((), ())),
                             preferred_element_type=jnp.float32)
        o_ref[...] = h2 * dv

    return pl.pallas_call(
        body,
        grid=(n // bs,),
        in_specs=[
            pl.BlockSpec((bs, d_hid), lambda i: (i, 0)),
            pl.BlockSpec((bs, d_hid), lambda i: (i, 0)),
            pl.BlockSpec((bs, 1), lambda i: (i, 0)),
            pl.BlockSpec((bs, 1), lambda i: (i, 0)),
            pl.BlockSpec((1, d_hid), lambda i: (0, 0)),
            pl.BlockSpec((d_out, d_hid), lambda i: (0, 0)),
        ],
        out_specs=pl.BlockSpec((bs, d_out), lambda i: (i, 0)),
        out_shape=jax.ShapeDtypeStruct((n, d_out), jnp.float32),
    )(p0, p1, deg0, deg1, b1, w2)


def _tc_layer3(q0, q1, deg0, deg1, b2, bs, d_real):
    n, d_pad = q0.shape

    def body(q0_ref, q1_ref, d0_ref, d1_ref, b_ref, o_ref):
        dv = lax.rsqrt(d0_ref[...] + d1_ref[...] + 1.0)
        z = ((q0_ref[...] + q1_ref[...]) * dv + b_ref[...])[:, :d_real]
        m = jnp.max(z, axis=1, keepdims=True)
        ez = jnp.exp(z - m)
        se = jnp.sum(ez, axis=1, keepdims=True)
        o_ref[...] = z - m - jnp.log(se)

    return pl.pallas_call(
        body,
        grid=(n // bs,),
        in_specs=[
            pl.BlockSpec((bs, d_pad), lambda i: (i, 0)),
            pl.BlockSpec((bs, d_pad), lambda i: (i, 0)),
            pl.BlockSpec((bs, 1), lambda i: (i, 0)),
            pl.BlockSpec((bs, 1), lambda i: (i, 0)),
            pl.BlockSpec((1, d_pad), lambda i: (0, 0)),
        ],
        out_specs=pl.BlockSpec((bs, d_real), lambda i: (i, 0)),
        out_shape=jax.ShapeDtypeStruct((n, d_real), jnp.float32),
    )(q0, q1, deg0, deg1, b2)


def kernel(x, edge_index, W1, b1, W2, b2):
    n, _ = x.shape
    d_hid = W1.shape[0]
    d_out = W2.shape[0]
    src = edge_index[0]
    dst = edge_index[1]
    e = src.shape[0]

    per_tile = -(-n // NS)
    per_tile += (-per_tile) % 16
    n_pad = per_tile * NS

    chunks = -(-e // (NW * CH))
    chunks += (-chunks) % 8
    e_pad = NW * CH * chunks
    pad = e_pad - e
    pad_ar = jnp.arange(pad, dtype=jnp.int32)
    src2d = jnp.concatenate([src, pad_ar % n]).reshape(NW * chunks, CH)
    dst2d = jnp.concatenate([dst, n + pad_ar % CH]).reshape(NW * chunks, CH)

    degp = _sc_degree(dst2d, n_pad)
    deg0 = degp[:n].reshape(n, 1)
    deg1 = degp[n_pad:n_pad + n].reshape(n, 1)

    bs = 1000
    h1 = _tc_layer1(x, W1, deg0, deg1, bs)
    p = _sc_aggregate(h1, jnp.zeros((n, d_hid), jnp.float32), src2d, dst2d)
    h2 = _tc_layer2(p[:n], p[n:], deg0, deg1, b1.reshape(1, d_hid), W2, bs)
    q = _sc_aggregate(h2, jnp.zeros((n, d_out), jnp.float32), src2d, dst2d,
                      tc_tiling=False)
    return _tc_layer3(q[:n], q[n:], deg0, deg1, b2.reshape(1, d_out), bs,
                      d_out)


# speedup vs baseline: 34.6462x; 1.0731x over previous
import functools

import jax
import jax.numpy as jnp
from jax import lax
from jax.experimental import pallas as pl
from jax.experimental.pallas import tpu as pltpu
from jax.experimental.pallas import tpu_sc as plsc

NC = 2
NS = 16
NW = NC * NS
CH = 128


def _mesh():
    return plsc.VectorSubcoreMesh(core_axis_name="c", subcore_axis_name="s")


def _sc_degree(dst2d, n_pad):
    chunks = dst2d.shape[0] // NW
    per_tile = n_pad // NS

    def body(dst_hbm, out_hbm, didx, ones_v, zb, acc):
        c = lax.axis_index("c")
        s = lax.axis_index("s")
        wid = s * NC + c

        def fill_ones(i, carry):
            ones_v[pl.ds(i * 16, 16)] = jnp.ones((16,), jnp.float32)
            return carry

        lax.fori_loop(0, CH // 16, fill_ones, 0)

        def fill_z(i, carry):
            zb[pl.ds(i * 16, 16)] = jnp.zeros((16,), jnp.float32)
            return carry

        lax.fori_loop(0, per_tile // 16, fill_z, 0)
        pltpu.sync_copy(zb, acc.at[pl.ds(s * per_tile, per_tile)])
        pltpu.sync_copy(dst_hbm.at[pl.ds(wid * chunks, chunks)], didx)
        plsc.subcore_barrier()

        def chunk(i, carry):
            pltpu.sync_copy(ones_v, acc.at[didx.at[i]], add=True)
            return carry

        lax.fori_loop(0, chunks, chunk, 0)
        plsc.subcore_barrier()
        pltpu.sync_copy(acc.at[pl.ds(s * per_tile, per_tile)],
                        out_hbm.at[pl.ds(c * n_pad + s * per_tile, per_tile)])

    return pl.kernel(
        body,
        out_type=jax.ShapeDtypeStruct((NC * n_pad,), jnp.float32),
        mesh=_mesh(),
        scratch_types=[
            pltpu.VMEM((chunks, CH), jnp.int32),
            pltpu.VMEM((CH,), jnp.float32),
            pltpu.VMEM((per_tile,), jnp.float32),
            pltpu.VMEM_SHARED((n_pad,), jnp.float32),
        ],
    )(dst2d)


def _sc_aggregate(h, src2d, dst2d, tc_tiling=True):
    n, d = h.shape
    chunks = src2d.shape[0] // NW
    ir = 1000
    ni = n // ir
    nstage = 1 if d <= 64 else 2
    hc = chunks // nstage

    def body(h_hbm, src_hbm, dst_hbm, out_hbm,
             sidx, didx, rows, acc, gs0, gs1):
        c = lax.axis_index("c")
        s = lax.axis_index("s")
        wid = s * NC + c
        rslice = pl.ds(s * ir, ir)

        @pl.when(s < ni)
        def _():
            pltpu.sync_copy(h_hbm.at[rslice], acc.at[rslice])

        plsc.subcore_barrier()

        for half in range(nstage):
            off = wid * chunks + half * hc
            pltpu.sync_copy(src_hbm.at[pl.ds(off, hc)], sidx)
            pltpu.sync_copy(dst_hbm.at[pl.ds(off, hc)], didx)
            pltpu.async_copy(h_hbm.at[sidx.at[0]], rows.at[0], gs0)
            pltpu.async_copy(h_hbm.at[sidx.at[1]], rows.at[1], gs1)

            def pair(g, carry):
                i0 = 2 * g
                pltpu.make_async_copy(h_hbm.at[sidx.at[i0]], rows.at[0],
                                      gs0).wait()
                pltpu.sync_copy(rows.at[0], acc.at[didx.at[i0]], add=True)

                @pl.when(i0 + 2 < hc)
                def _():
                    pltpu.async_copy(h_hbm.at[sidx.at[i0 + 2]], rows.at[0],
                                     gs0)

                pltpu.make_async_copy(h_hbm.at[sidx.at[i0 + 1]], rows.at[1],
                                      gs1).wait()
                pltpu.sync_copy(rows.at[1], acc.at[didx.at[i0 + 1]], add=True)

                @pl.when(i0 + 3 < hc)
                def _():
                    pltpu.async_copy(h_hbm.at[sidx.at[i0 + 3]], rows.at[1],
                                     gs1)

                return carry

            lax.fori_loop(0, hc // 2, pair, 0)
        plsc.subcore_barrier()

        @pl.when(s < ni)
        def _():
            pltpu.sync_copy(acc.at[rslice],
                            out_hbm.at[pl.ds(c * n + s * ir, ir)])

    return pl.kernel(
        body,
        out_type=jax.ShapeDtypeStruct((NC * n, d), jnp.float32),
        mesh=_mesh(),
        scratch_types=[
            pltpu.VMEM((hc, CH), jnp.int32),
            pltpu.VMEM((hc, CH), jnp.int32),
            pltpu.VMEM((2, CH, d), jnp.float32),
            pltpu.VMEM_SHARED((n + CH, d), jnp.float32),
            pltpu.SemaphoreType.DMA,
            pltpu.SemaphoreType.DMA,
        ],
        compiler_params=pltpu.CompilerParams(use_tc_tiling_on_sc=tc_tiling),
    )(h, src2d, dst2d)


def _tc_layer1(x, w1, deg0, deg1, bs):
    n, d_in = x.shape
    d_hid = w1.shape[0]

    def body(x_ref, w_ref, d0_ref, d1_ref, o_ref):
        dv = lax.rsqrt(d0_ref[...] + d1_ref[...] + 1.0)
        hm = lax.dot_general(x_ref[...], w_ref[...], (((1,), (1,)), ((), ())),
                             preferred_element_type=jnp.float32)
        o_ref[...] = hm * dv

    return pl.pallas_call(
        body,
        grid=(n // bs,),
        in_specs=[
            pl.BlockSpec((bs, d_in), lambda i: (i, 0)),
            pl.BlockSpec((d_hid, d_in), lambda i: (0, 0)),
            pl.BlockSpec((bs, 1), lambda i: (i, 0)),
            pl.BlockSpec((bs, 1), lambda i: (i, 0)),
        ],
        out_specs=pl.BlockSpec((bs, d_hid), lambda i: (i, 0)),
        out_shape=jax.ShapeDtypeStruct((n, d_hid), jnp.float32),
    )(x, w1, deg0, deg1)


def _tc_layer2(p, h1, deg0, deg1, b1, w2, bs):
    d_hid = h1.shape[1]
    n = p.shape[0] // NC
    d_out = w2.shape[0]
    gn = n // bs

    def body(p0_ref, p1_ref, h_ref, d0_ref, d1_ref, b_ref, w_ref, o_ref):
        dv = lax.rsqrt(d0_ref[...] + d1_ref[...] + 1.0)
        agg = p0_ref[...] + p1_ref[...] - h_ref[...]
        x1 = jnp.maximum(agg * dv + b_ref[...], 0.0)
        h2 = lax.dot_general(x1, w_ref[...], (((1,), (1,)), ((), ())),
                             preferred_element_type=jnp.float32)
        o_ref[...] = h2 * dv

    return pl.pallas_call(
        body,
        grid=(gn,),
        in_specs=[
            pl.BlockSpec((bs, d_hid), lambda i: (i, 0)),
            pl.BlockSpec((bs, d_hid), lambda i: (i + gn, 0)),
            pl.BlockSpec((bs, d_hid), lambda i: (i, 0)),
            pl.BlockSpec((bs, 1), lambda i: (i, 0)),
            pl.BlockSpec((bs, 1), lambda i: (i, 0)),
            pl.BlockSpec((1, d_hid), lambda i: (0, 0)),
            pl.BlockSpec((d_out, d_hid), lambda i: (0, 0)),
        ],
        out_specs=pl.BlockSpec((bs, d_out), lambda i: (i, 0)),
        out_shape=jax.ShapeDtypeStruct((n, d_out), jnp.float32),
    )(p, p, h1, deg0, deg1, b1, w2)


def _tc_layer3(q, h2, deg0, deg1, b2, bs):
    d_out = h2.shape[1]
    n = q.shape[0] // NC
    gn = n // bs

    def body(q0_ref, q1_ref, h_ref, d0_ref, d1_ref, b_ref, o_ref):
        dv = lax.rsqrt(d0_ref[...] + d1_ref[...] + 1.0)
        agg = q0_ref[...] + q1_ref[...] - h_ref[...]
        z = agg * dv + b_ref[...]
        m = jnp.max(z, axis=1, keepdims=True)
        ez = jnp.exp(z - m)
        se = jnp.sum(ez, axis=1, keepdims=True)
        o_ref[...] = z - m - jnp.log(se)

    return pl.pallas_call(
        body,
        grid=(gn,),
        in_specs=[
            pl.BlockSpec((bs, d_out), lambda i: (i, 0)),
            pl.BlockSpec((bs, d_out), lambda i: (i + gn, 0)),
            pl.BlockSpec((bs, d_out), lambda i: (i, 0)),
            pl.BlockSpec((bs, 1), lambda i: (i, 0)),
            pl.BlockSpec((bs, 1), lambda i: (i, 0)),
            pl.BlockSpec((1, d_out), lambda i: (0, 0)),
        ],
        out_specs=pl.BlockSpec((bs, d_out), lambda i: (i, 0)),
        out_shape=jax.ShapeDtypeStruct((n, d_out), jnp.float32),
    )(q, q, h2, deg0, deg1, b2)


def kernel(x, edge_index, W1, b1, W2, b2):
    n, _ = x.shape
    d_hid = W1.shape[0]
    d_out = W2.shape[0]
    src = edge_index[0]
    dst = edge_index[1]
    e = src.shape[0]

    per_tile = -(-n // NS)
    per_tile += (-per_tile) % 16
    n_pad = per_tile * NS

    chunks = -(-e // (NW * CH))
    chunks += (-chunks) % 8
    e_pad = NW * CH * chunks
    pad = e_pad - e
    pad_ar = jnp.arange(pad, dtype=jnp.int32)
    src2d = jnp.concatenate([src, pad_ar % n]).reshape(NW * chunks, CH)
    dst2d = jnp.concatenate([dst, n + pad_ar % CH]).reshape(NW * chunks, CH)

    degp = _sc_degree(dst2d, n_pad)
    deg0 = degp[:n].reshape(n, 1)
    deg1 = degp[n_pad:n_pad + n].reshape(n, 1)

    bs = 2000
    h1 = _tc_layer1(x, W1, deg0, deg1, bs)
    p = _sc_aggregate(h1, src2d, dst2d)
    h2 = _tc_layer2(p, h1, deg0, deg1, b1.reshape(1, d_hid), W2, bs)
    q = _sc_aggregate(h2, src2d, dst2d, tc_tiling=False)
    return _tc_layer3(q, h2, deg0, deg1, b2.reshape(1, d_out), bs)


# speedup vs baseline: 34.7360x; 1.0026x over previous
import functools

import jax
import jax.numpy as jnp
from jax import lax
from jax.experimental import pallas as pl
from jax.experimental.pallas import tpu as pltpu
from jax.experimental.pallas import tpu_sc as plsc

NC = 2
NS = 16
NW = NC * NS
CH = 128


def _mesh():
    return plsc.VectorSubcoreMesh(core_axis_name="c", subcore_axis_name="s")


def _sc_degree(dst, n_pad):
    e = dst.shape[0]
    per_w = e // NW
    full = per_w // CH
    rem = per_w - full * CH
    per_tile = n_pad // NS

    def body(dst_hbm, out_hbm, didx, ones_v, zb, acc):
        c = lax.axis_index("c")
        s = lax.axis_index("s")
        wid = s * NC + c

        def fill_ones(i, carry):
            ones_v[pl.ds(i * 16, 16)] = jnp.ones((16,), jnp.float32)
            return carry

        lax.fori_loop(0, CH // 16, fill_ones, 0)

        def fill_z(i, carry):
            zb[pl.ds(i * 16, 16)] = jnp.zeros((16,), jnp.float32)
            return carry

        lax.fori_loop(0, per_tile // 16, fill_z, 0)
        pltpu.sync_copy(zb, acc.at[pl.ds(s * per_tile, per_tile)])
        pltpu.sync_copy(dst_hbm.at[pl.ds(wid * per_w, per_w)], didx)
        plsc.subcore_barrier()

        def chunk(i, carry):
            pltpu.sync_copy(ones_v, acc.at[didx.at[pl.ds(i * CH, CH)]],
                            add=True)
            return carry

        lax.fori_loop(0, full, chunk, 0)
        if rem:
            pltpu.sync_copy(ones_v.at[pl.ds(0, rem)],
                            acc.at[didx.at[pl.ds(full * CH, rem)]], add=True)
        plsc.subcore_barrier()
        pltpu.sync_copy(acc.at[pl.ds(s * per_tile, per_tile)],
                        out_hbm.at[pl.ds(c * n_pad + s * per_tile, per_tile)])

    return pl.kernel(
        body,
        out_type=jax.ShapeDtypeStruct((NC * n_pad,), jnp.float32),
        mesh=_mesh(),
        scratch_types=[
            pltpu.VMEM((per_w,), jnp.int32),
            pltpu.VMEM((CH,), jnp.float32),
            pltpu.VMEM((per_tile,), jnp.float32),
            pltpu.VMEM_SHARED((n_pad,), jnp.float32),
        ],
    )(dst)


def _sc_aggregate(h, src, dst, tc_tiling=True):
    n, d = h.shape
    e = src.shape[0]
    per_w = e // NW
    full = per_w // CH
    rem = per_w - full * CH
    ir = 1000
    ni = n // ir
    if d <= 64:
        stages = [(full, 0)]
    else:
        s0 = -(-full // 2)
        s0 += s0 % 2
        stages = [(s0, 0), (full - s0, s0)]
    buf_c = stages[0][0]

    def body(h_hbm, src_hbm, dst_hbm, out_hbm,
             sidx, didx, sidx_r, didx_r, rows, rows_r, acc, gs0, gs1):
        c = lax.axis_index("c")
        s = lax.axis_index("s")
        wid = s * NC + c
        rslice = pl.ds(s * ir, ir)
        base0 = wid * per_w

        @pl.when(s < ni)
        def _():
            pltpu.sync_copy(h_hbm.at[rslice], acc.at[rslice])

        if rem:
            pltpu.sync_copy(src_hbm.at[pl.ds(base0 + full * CH, rem)], sidx_r)
            pltpu.sync_copy(dst_hbm.at[pl.ds(base0 + full * CH, rem)], didx_r)
        plsc.subcore_barrier()

        for sc, off_c in stages:
            off = base0 + off_c * CH
            pltpu.sync_copy(src_hbm.at[pl.ds(off, sc * CH)],
                            sidx.at[pl.ds(0, sc * CH)])
            pltpu.sync_copy(dst_hbm.at[pl.ds(off, sc * CH)],
                            didx.at[pl.ds(0, sc * CH)])
            pltpu.async_copy(h_hbm.at[sidx.at[pl.ds(0, CH)]], rows.at[0], gs0)
            pltpu.async_copy(h_hbm.at[sidx.at[pl.ds(CH, CH)]], rows.at[1],
                             gs1)

            def pair(g, carry):
                i0 = 2 * g
                pltpu.make_async_copy(h_hbm.at[sidx.at[pl.ds(i0 * CH, CH)]],
                                      rows.at[0], gs0).wait()
                pltpu.sync_copy(rows.at[0],
                                acc.at[didx.at[pl.ds(i0 * CH, CH)]], add=True)

                @pl.when(i0 + 2 < sc)
                def _():
                    pltpu.async_copy(
                        h_hbm.at[sidx.at[pl.ds((i0 + 2) * CH, CH)]],
                        rows.at[0], gs0)

                pltpu.make_async_copy(
                    h_hbm.at[sidx.at[pl.ds((i0 + 1) * CH, CH)]],
                    rows.at[1], gs1).wait()
                pltpu.sync_copy(rows.at[1],
                                acc.at[didx.at[pl.ds((i0 + 1) * CH, CH)]],
                                add=True)

                @pl.when(i0 + 3 < sc)
                def _():
                    pltpu.async_copy(
                        h_hbm.at[sidx.at[pl.ds((i0 + 3) * CH, CH)]],
                        rows.at[1], gs1)

                return carry

            lax.fori_loop(0, sc // 2, pair, 0)

        if rem:
            pltpu.async_copy(h_hbm.at[sidx_r], rows_r, gs0).wait()
            pltpu.sync_copy(rows_r, acc.at[didx_r], add=True)
        plsc.subcore_barrier()

        @pl.when(s < ni)
        def _():
            pltpu.sync_copy(acc.at[rslice],
                            out_hbm.at[pl.ds(c * n + s * ir, ir)])

    return pl.kernel(
        body,
        out_type=jax.ShapeDtypeStruct((NC * n, d), jnp.float32),
        mesh=_mesh(),
        scratch_types=[
            pltpu.VMEM((buf_c * CH,), jnp.int32),
            pltpu.VMEM((buf_c * CH,), jnp.int32),
            pltpu.VMEM((max(rem, 16),), jnp.int32),
            pltpu.VMEM((max(rem, 16),), jnp.int32),
            pltpu.VMEM((2, CH, d), jnp.float32),
            pltpu.VMEM((max(rem, 16), d), jnp.float32),
            pltpu.VMEM_SHARED((n, d), jnp.float32),
            pltpu.SemaphoreType.DMA,
            pltpu.SemaphoreType.DMA,
        ],
        compiler_params=pltpu.CompilerParams(use_tc_tiling_on_sc=tc_tiling),
    )(h, src, dst)


def _tc_layer1(x, w1, deg0, deg1, bs):
    n, d_in = x.shape
    d_hid = w1.shape[0]

    def body(x_ref, w_ref, d0_ref, d1_ref, o_ref):
        dv = lax.rsqrt(d0_ref[...] + d1_ref[...] + 1.0)
        hm = lax.dot_general(x_ref[...], w_ref[...], (((1,), (1,)), ((), ())),
                             preferred_element_type=jnp.float32)
        o_ref[...] = hm * dv

    return pl.pallas_call(
        body,
        grid=(n // bs,),
        in_specs=[
            pl.BlockSpec((bs, d_in), lambda i: (i, 0)),
            pl.BlockSpec((d_hid, d_in), lambda i: (0, 0)),
            pl.BlockSpec((bs, 1), lambda i: (i, 0)),
            pl.BlockSpec((bs, 1), lambda i: (i, 0)),
        ],
        out_specs=pl.BlockSpec((bs, d_hid), lambda i: (i, 0)),
        out_shape=jax.ShapeDtypeStruct((n, d_hid), jnp.float32),
    )(x, w1, deg0, deg1)


def _tc_layer2(p, h1, deg0, deg1, b1, w2, bs):
    d_hid = h1.shape[1]
    n = p.shape[0] // NC
    d_out = w2.shape[0]
    gn = n // bs

    def body(p0_ref, p1_ref, h_ref, d0_ref, d1_ref, b_ref, w_ref, o_ref):
        dv = lax.rsqrt(d0_ref[...] + d1_ref[...] + 1.0)
        agg = p0_ref[...] + p1_ref[...] - h_ref[...]
        x1 = jnp.maximum(agg * dv + b_ref[...], 0.0)
        h2 = lax.dot_general(x1, w_ref[...], (((1,), (1,)), ((), ())),
                             preferred_element_type=jnp.float32)
        o_ref[...] = h2 * dv

    return pl.pallas_call(
        body,
        grid=(gn,),
        in_specs=[
            pl.BlockSpec((bs, d_hid), lambda i: (i, 0)),
            pl.BlockSpec((bs, d_hid), lambda i: (i + gn, 0)),
            pl.BlockSpec((bs, d_hid), lambda i: (i, 0)),
            pl.BlockSpec((bs, 1), lambda i: (i, 0)),
            pl.BlockSpec((bs, 1), lambda i: (i, 0)),
            pl.BlockSpec((1, d_hid), lambda i: (0, 0)),
            pl.BlockSpec((d_out, d_hid), lambda i: (0, 0)),
        ],
        out_specs=pl.BlockSpec((bs, d_out), lambda i: (i, 0)),
        out_shape=jax.ShapeDtypeStruct((n, d_out), jnp.float32),
    )(p, p, h1, deg0, deg1, b1, w2)


def _tc_layer3(q, h2, deg0, deg1, b2, bs):
    d_out = h2.shape[1]
    n = q.shape[0] // NC
    gn = n // bs

    def body(q0_ref, q1_ref, h_ref, d0_ref, d1_ref, b_ref, o_ref):
        dv = lax.rsqrt(d0_ref[...] + d1_ref[...] + 1.0)
        agg = q0_ref[...] + q1_ref[...] - h_ref[...]
        z = agg * dv + b_ref[...]
        m = jnp.max(z, axis=1, keepdims=True)
        ez = jnp.exp(z - m)
        se = jnp.sum(ez, axis=1, keepdims=True)
        o_ref[...] = z - m - jnp.log(se)

    return pl.pallas_call(
        body,
        grid=(gn,),
        in_specs=[
            pl.BlockSpec((bs, d_out), lambda i: (i, 0)),
            pl.BlockSpec((bs, d_out), lambda i: (i + gn, 0)),
            pl.BlockSpec((bs, d_out), lambda i: (i, 0)),
            pl.BlockSpec((bs, 1), lambda i: (i, 0)),
            pl.BlockSpec((bs, 1), lambda i: (i, 0)),
            pl.BlockSpec((1, d_out), lambda i: (0, 0)),
        ],
        out_specs=pl.BlockSpec((bs, d_out), lambda i: (i, 0)),
        out_shape=jax.ShapeDtypeStruct((n, d_out), jnp.float32),
    )(q, q, h2, deg0, deg1, b2)


def kernel(x, edge_index, W1, b1, W2, b2):
    n, _ = x.shape
    d_hid = W1.shape[0]
    d_out = W2.shape[0]
    src = edge_index[0]
    dst = edge_index[1]

    per_tile = -(-n // NS)
    per_tile += (-per_tile) % 16
    n_pad = per_tile * NS

    degp = _sc_degree(dst, n_pad)
    deg0 = degp[:n].reshape(n, 1)
    deg1 = degp[n_pad:n_pad + n].reshape(n, 1)

    bs = 2000
    h1 = _tc_layer1(x, W1, deg0, deg1, bs)
    p = _sc_aggregate(h1, src, dst)
    h2 = _tc_layer2(p, h1, deg0, deg1, b1.reshape(1, d_hid), W2, bs)
    q = _sc_aggregate(h2, src, dst, tc_tiling=False)
    return _tc_layer3(q, h2, deg0, deg1, b2.reshape(1, d_out), bs)


# speedup vs baseline: 36.1766x; 1.0415x over previous
import functools

import jax
import jax.numpy as jnp
from jax import lax
from jax.experimental import pallas as pl
from jax.experimental.pallas import tpu as pltpu
from jax.experimental.pallas import tpu_sc as plsc

NC = 2
NS = 16
NW = NC * NS
CH = 128


def _mesh():
    return plsc.VectorSubcoreMesh(core_axis_name="c", subcore_axis_name="s")


def _sc_degree(ei, e, n_pad):
    per_w = e // NW
    full = per_w // CH
    rem = per_w - full * CH
    per_tile = n_pad // NS

    def body(ei_hbm, out_hbm, didx, ones_v, zb, acc):
        c = lax.axis_index("c")
        s = lax.axis_index("s")
        wid = s * NC + c

        def fill_ones(i, carry):
            ones_v[pl.ds(i * 16, 16)] = jnp.ones((16,), jnp.float32)
            return carry

        lax.fori_loop(0, CH // 16, fill_ones, 0)

        def fill_z(i, carry):
            zb[pl.ds(i * 16, 16)] = jnp.zeros((16,), jnp.float32)
            return carry

        lax.fori_loop(0, per_tile // 16, fill_z, 0)
        pltpu.sync_copy(zb, acc.at[pl.ds(s * per_tile, per_tile)])
        pltpu.sync_copy(ei_hbm.at[pl.ds(e + wid * per_w, per_w)], didx)
        plsc.subcore_barrier()

        def chunk(i, carry):
            pltpu.sync_copy(ones_v, acc.at[didx.at[pl.ds(i * CH, CH)]],
                            add=True)
            return carry

        lax.fori_loop(0, full, chunk, 0)
        if rem:
            pltpu.sync_copy(ones_v.at[pl.ds(0, rem)],
                            acc.at[didx.at[pl.ds(full * CH, rem)]], add=True)
        plsc.subcore_barrier()
        pltpu.sync_copy(acc.at[pl.ds(s * per_tile, per_tile)],
                        out_hbm.at[pl.ds(c * n_pad + s * per_tile, per_tile)])

    return pl.kernel(
        body,
        out_type=jax.ShapeDtypeStruct((NC * n_pad,), jnp.float32),
        mesh=_mesh(),
        scratch_types=[
            pltpu.VMEM((per_w,), jnp.int32),
            pltpu.VMEM((CH,), jnp.float32),
            pltpu.VMEM((per_tile,), jnp.float32),
            pltpu.VMEM_SHARED((n_pad,), jnp.float32),
        ],
    )(ei)


def _sc_aggregate(h, ei, e, tc_tiling=True):
    n, d = h.shape
    per_w = e // NW
    full = per_w // CH
    rem = per_w - full * CH
    ir = 1000
    ni = n // ir
    if d <= 64:
        stages = [(full, 0)]
    else:
        s0 = -(-full // 2)
        s0 += s0 % 2
        stages = [(s0, 0), (full - s0, s0)]
    buf_c = stages[0][0]

    def body(h_hbm, ei_hbm, out_hbm,
             sidx, didx, sidx_r, didx_r, rows, rows_r, acc, gs0, gs1):
        c = lax.axis_index("c")
        s = lax.axis_index("s")
        wid = s * NC + c
        rslice = pl.ds(s * ir, ir)
        base0 = wid * per_w

        @pl.when(s < ni)
        def _():
            pltpu.sync_copy(h_hbm.at[rslice], acc.at[rslice])

        if rem:
            pltpu.sync_copy(ei_hbm.at[pl.ds(base0 + full * CH, rem)], sidx_r)
            pltpu.sync_copy(ei_hbm.at[pl.ds(e + base0 + full * CH, rem)],
                            didx_r)
        plsc.subcore_barrier()

        for sc, off_c in stages:
            off = base0 + off_c * CH
            pltpu.sync_copy(ei_hbm.at[pl.ds(off, sc * CH)],
                            sidx.at[pl.ds(0, sc * CH)])
            pltpu.sync_copy(ei_hbm.at[pl.ds(e + off, sc * CH)],
                            didx.at[pl.ds(0, sc * CH)])
            pltpu.async_copy(h_hbm.at[sidx.at[pl.ds(0, CH)]], rows.at[0], gs0)
            pltpu.async_copy(h_hbm.at[sidx.at[pl.ds(CH, CH)]], rows.at[1],
                             gs1)

            def pair(g, carry):
                i0 = 2 * g
                pltpu.make_async_copy(h_hbm.at[sidx.at[pl.ds(i0 * CH, CH)]],
                                      rows.at[0], gs0).wait()
                pltpu.sync_copy(rows.at[0],
                                acc.at[didx.at[pl.ds(i0 * CH, CH)]], add=True)

                @pl.when(i0 + 2 < sc)
                def _():
                    pltpu.async_copy(
                        h_hbm.at[sidx.at[pl.ds((i0 + 2) * CH, CH)]],
                        rows.at[0], gs0)

                pltpu.make_async_copy(
                    h_hbm.at[sidx.at[pl.ds((i0 + 1) * CH, CH)]],
                    rows.at[1], gs1).wait()
                pltpu.sync_copy(rows.at[1],
                                acc.at[didx.at[pl.ds((i0 + 1) * CH, CH)]],
                                add=True)

                @pl.when(i0 + 3 < sc)
                def _():
                    pltpu.async_copy(
                        h_hbm.at[sidx.at[pl.ds((i0 + 3) * CH, CH)]],
                        rows.at[1], gs1)

                return carry

            lax.fori_loop(0, sc // 2, pair, 0)

        if rem:
            pltpu.async_copy(h_hbm.at[sidx_r], rows_r, gs0).wait()
            pltpu.sync_copy(rows_r, acc.at[didx_r], add=True)
        plsc.subcore_barrier()

        @pl.when(s < ni)
        def _():
            pltpu.sync_copy(acc.at[rslice],
                            out_hbm.at[pl.ds(c * n + s * ir, ir)])

    return pl.kernel(
        body,
        out_type=jax.ShapeDtypeStruct((NC * n, d), jnp.float32),
        mesh=_mesh(),
        scratch_types=[
            pltpu.VMEM((buf_c * CH,), jnp.int32),
            pltpu.VMEM((buf_c * CH,), jnp.int32),
            pltpu.VMEM((max(rem, 16),), jnp.int32),
            pltpu.VMEM((max(rem, 16),), jnp.int32),
            pltpu.VMEM((2, CH, d), jnp.float32),
            pltpu.VMEM((max(rem, 16), d), jnp.float32),
            pltpu.VMEM_SHARED((n, d), jnp.float32),
            pltpu.SemaphoreType.DMA,
            pltpu.SemaphoreType.DMA,
        ],
        compiler_params=pltpu.CompilerParams(use_tc_tiling_on_sc=tc_tiling),
    )(h, ei)


def _tc_layer1(x, w1, deg0, deg1, bs):
    n, d_in = x.shape
    d_hid = w1.shape[0]

    def body(x_ref, w_ref, d0_ref, d1_ref, o_ref):
        dv = lax.rsqrt(d0_ref[...] + d1_ref[...] + 1.0)
        hm = lax.dot_general(x_ref[...], w_ref[...], (((1,), (1,)), ((), ())),
                             preferred_element_type=jnp.float32)
        o_ref[...] = hm * dv

    return pl.pallas_call(
        body,
        grid=(n // bs,),
        in_specs=[
            pl.BlockSpec((bs, d_in), lambda i: (i, 0)),
            pl.BlockSpec((d_hid, d_in), lambda i: (0, 0)),
            pl.BlockSpec((bs, 1), lambda i: (i, 0)),
            pl.BlockSpec((bs, 1), lambda i: (i, 0)),
        ],
        out_specs=pl.BlockSpec((bs, d_hid), lambda i: (i, 0)),
        out_shape=jax.ShapeDtypeStruct((n, d_hid), jnp.float32),
    )(x, w1, deg0, deg1)


def _tc_layer2(p, h1, deg0, deg1, b1, w2, bs):
    d_hid = h1.shape[1]
    n = p.shape[0] // NC
    d_out = w2.shape[0]
    gn = n // bs

    def body(p0_ref, p1_ref, h_ref, d0_ref, d1_ref, b_ref, w_ref, o_ref):
        dv = lax.rsqrt(d0_ref[...] + d1_ref[...] + 1.0)
        agg = p0_ref[...] + p1_ref[...] - h_ref[...]
        x1 = jnp.maximum(agg * dv + b_ref[...], 0.0)
        h2 = lax.dot_general(x1, w_ref[...], (((1,), (1,)), ((), ())),
                             preferred_element_type=jnp.float32)
        o_ref[...] = h2 * dv

    return pl.pallas_call(
        body,
        grid=(gn,),
        in_specs=[
            pl.BlockSpec((bs, d_hid), lambda i: (i, 0)),
            pl.BlockSpec((bs, d_hid), lambda i: (i + gn, 0)),
            pl.BlockSpec((bs, d_hid), lambda i: (i, 0)),
            pl.BlockSpec((bs, 1), lambda i: (i, 0)),
            pl.BlockSpec((bs, 1), lambda i: (i, 0)),
            pl.BlockSpec((1, d_hid), lambda i: (0, 0)),
            pl.BlockSpec((d_out, d_hid), lambda i: (0, 0)),
        ],
        out_specs=pl.BlockSpec((bs, d_out), lambda i: (i, 0)),
        out_shape=jax.ShapeDtypeStruct((n, d_out), jnp.float32),
    )(p, p, h1, deg0, deg1, b1, w2)


def _tc_layer3(q, h2, deg0, deg1, b2, bs):
    d_out = h2.shape[1]
    n = q.shape[0] // NC
    gn = n // bs

    def body(q0_ref, q1_ref, h_ref, d0_ref, d1_ref, b_ref, o_ref):
        dv = lax.rsqrt(d0_ref[...] + d1_ref[...] + 1.0)
        agg = q0_ref[...] + q1_ref[...] - h_ref[...]
        z = agg * dv + b_ref[...]
        m = jnp.max(z, axis=1, keepdims=True)
        ez = jnp.exp(z - m)
        se = jnp.sum(ez, axis=1, keepdims=True)
        o_ref[...] = z - m - jnp.log(se)

    return pl.pallas_call(
        body,
        grid=(gn,),
        in_specs=[
            pl.BlockSpec((bs, d_out), lambda i: (i, 0)),
            pl.BlockSpec((bs, d_out), lambda i: (i + gn, 0)),
            pl.BlockSpec((bs, d_out), lambda i: (i, 0)),
            pl.BlockSpec((bs, 1), lambda i: (i, 0)),
            pl.BlockSpec((bs, 1), lambda i: (i, 0)),
            pl.BlockSpec((1, d_out), lambda i: (0, 0)),
        ],
        out_specs=pl.BlockSpec((bs, d_out), lambda i: (i, 0)),
        out_shape=jax.ShapeDtypeStruct((n, d_out), jnp.float32),
    )(q, q, h2, deg0, deg1, b2)


def kernel(x, edge_index, W1, b1, W2, b2):
    n, _ = x.shape
    d_hid = W1.shape[0]
    d_out = W2.shape[0]
    e = edge_index.shape[1]
    ei = edge_index.reshape(2 * e)

    per_tile = -(-n // NS)
    per_tile += (-per_tile) % 16
    n_pad = per_tile * NS

    degp = _sc_degree(ei, e, n_pad)
    deg0 = degp[:n].reshape(n, 1)
    deg1 = degp[n_pad:n_pad + n].reshape(n, 1)

    bs = 2000
    h1 = _tc_layer1(x, W1, deg0, deg1, bs)
    p = _sc_aggregate(h1, ei, e)
    h2 = _tc_layer2(p, h1, deg0, deg1, b1.reshape(1, d_hid), W2, bs)
    q = _sc_aggregate(h2, ei, e, tc_tiling=False)
    return _tc_layer3(q, h2, deg0, deg1, b2.reshape(1, d_out), bs)


# speedup vs baseline: 37.2971x; 1.0310x over previous
import functools

import jax
import jax.numpy as jnp
from jax import lax
from jax.experimental import pallas as pl
from jax.experimental.pallas import tpu as pltpu
from jax.experimental.pallas import tpu_sc as plsc

NC = 2
NS = 16
NW = NC * NS
CH = 128


def _mesh():
    return plsc.VectorSubcoreMesh(core_axis_name="c", subcore_axis_name="s")


def _sc_degree(ei, e, n_pad):
    per_w = e // NW
    full = per_w // CH
    rem = per_w - full * CH
    per_tile = n_pad // NS

    def body(ei_hbm, out_hbm, didx, ones_v, zb, acc, hs):
        c = lax.axis_index("c")
        s = lax.axis_index("s")
        wid = s * NC + c

        def fill_ones(i, carry):
            ones_v[pl.ds(i * 16, 16)] = jnp.ones((16,), jnp.float32)
            return carry

        lax.fori_loop(0, CH // 16, fill_ones, 0)

        def fill_z(i, carry):
            zb[pl.ds(i * 16, 16)] = jnp.zeros((16,), jnp.float32)
            return carry

        lax.fori_loop(0, per_tile // 16, fill_z, 0)
        pltpu.sync_copy(zb, acc.at[pl.ds(s * per_tile, per_tile)])
        pltpu.sync_copy(ei_hbm.at[pl.ds(e + wid * per_w, per_w)], didx)
        plsc.subcore_barrier()

        def chunk(i, carry):
            pltpu.async_copy(ones_v, acc.at[didx.at[pl.ds(i * CH, CH)]], hs,
                             add=True)
            return carry

        lax.fori_loop(0, full, chunk, 0)

        def drain(i, carry):
            pltpu.make_async_copy(ones_v,
                                  acc.at[didx.at[pl.ds(i * CH, CH)]],
                                  hs).wait()
            return carry

        lax.fori_loop(0, full, drain, 0)
        if rem:
            pltpu.sync_copy(ones_v.at[pl.ds(0, rem)],
                            acc.at[didx.at[pl.ds(full * CH, rem)]], add=True)
        plsc.subcore_barrier()
        pltpu.sync_copy(acc.at[pl.ds(s * per_tile, per_tile)],
                        out_hbm.at[pl.ds(c * n_pad + s * per_tile, per_tile)])

    return pl.kernel(
        body,
        out_type=jax.ShapeDtypeStruct((NC * n_pad,), jnp.float32),
        mesh=_mesh(),
        scratch_types=[
            pltpu.VMEM((per_w,), jnp.int32),
            pltpu.VMEM((CH,), jnp.float32),
            pltpu.VMEM((per_tile,), jnp.float32),
            pltpu.VMEM_SHARED((n_pad,), jnp.float32),
            pltpu.SemaphoreType.DMA,
        ],
    )(ei)


def _sc_aggregate(h, ei, e, tc_tiling=True):
    n, d = h.shape
    per_w = e // NW
    full = per_w // CH
    rem = per_w - full * CH
    ir = 1000
    ni = n // ir
    if d <= 64:
        stages = [(full, 0)]
    else:
        s0 = -(-full // 2)
        s0 += s0 % 2
        stages = [(s0, 0), (full - s0, s0)]
    buf_c = stages[0][0]

    def body(h_hbm, ei_hbm, out_hbm,
             sidx, didx, sidx_r, didx_r, rows, rows_r, acc, gs0, gs1):
        c = lax.axis_index("c")
        s = lax.axis_index("s")
        wid = s * NC + c
        rslice = pl.ds(s * ir, ir)
        base0 = wid * per_w

        def stage_in(sc, off_c):
            off = base0 + off_c * CH
            pltpu.sync_copy(ei_hbm.at[pl.ds(off, sc * CH)],
                            sidx.at[pl.ds(0, sc * CH)])
            pltpu.sync_copy(ei_hbm.at[pl.ds(e + off, sc * CH)],
                            didx.at[pl.ds(0, sc * CH)])
            pltpu.async_copy(h_hbm.at[sidx.at[pl.ds(0, CH)]], rows.at[0], gs0)
            pltpu.async_copy(h_hbm.at[sidx.at[pl.ds(CH, CH)]], rows.at[1],
                             gs1)

        stage_in(*stages[0])
        if rem:
            pltpu.sync_copy(ei_hbm.at[pl.ds(base0 + full * CH, rem)], sidx_r)
            pltpu.sync_copy(ei_hbm.at[pl.ds(e + base0 + full * CH, rem)],
                            didx_r)

        @pl.when(s < ni)
        def _():
            pltpu.sync_copy(h_hbm.at[rslice], acc.at[rslice])

        plsc.subcore_barrier()

        for si, (sc, off_c) in enumerate(stages):
            if si:
                stage_in(sc, off_c)

            def pair(g, carry):
                i0 = 2 * g
                pltpu.make_async_copy(h_hbm.at[sidx.at[pl.ds(i0 * CH, CH)]],
                                      rows.at[0], gs0).wait()
                pltpu.sync_copy(rows.at[0],
                                acc.at[didx.at[pl.ds(i0 * CH, CH)]], add=True)

                @pl.when(i0 + 2 < sc)
                def _():
                    pltpu.async_copy(
                        h_hbm.at[sidx.at[pl.ds((i0 + 2) * CH, CH)]],
                        rows.at[0], gs0)

                pltpu.make_async_copy(
                    h_hbm.at[sidx.at[pl.ds((i0 + 1) * CH, CH)]],
                    rows.at[1], gs1).wait()
                pltpu.sync_copy(rows.at[1],
                                acc.at[didx.at[pl.ds((i0 + 1) * CH, CH)]],
                                add=True)

                @pl.when(i0 + 3 < sc)
                def _():
                    pltpu.async_copy(
                        h_hbm.at[sidx.at[pl.ds((i0 + 3) * CH, CH)]],
                        rows.at[1], gs1)

                return carry

            lax.fori_loop(0, sc // 2, pair, 0)

        if rem:
            pltpu.async_copy(h_hbm.at[sidx_r], rows_r, gs0).wait()
            pltpu.sync_copy(rows_r, acc.at[didx_r], add=True)
        plsc.subcore_barrier()

        @pl.when(s < ni)
        def _():
            pltpu.sync_copy(acc.at[rslice],
                            out_hbm.at[pl.ds(c * n + s * ir, ir)])

    return pl.kernel(
        body,
        out_type=jax.ShapeDtypeStruct((NC * n, d), jnp.float32),
        mesh=_mesh(),
        scratch_types=[
            pltpu.VMEM((buf_c * CH,), jnp.int32),
            pltpu.VMEM((buf_c * CH,), jnp.int32),
            pltpu.VMEM((max(rem, 16),), jnp.int32),
            pltpu.VMEM((max(rem, 16),), jnp.int32),
            pltpu.VMEM((2, CH, d), jnp.float32),
            pltpu.VMEM((max(rem, 16), d), jnp.float32),
            pltpu.VMEM_SHARED((n, d), jnp.float32),
            pltpu.SemaphoreType.DMA,
            pltpu.SemaphoreType.DMA,
        ],
        compiler_params=pltpu.CompilerParams(use_tc_tiling_on_sc=tc_tiling),
    )(h, ei)


def _tc_layer1(x, w1, deg0, deg1, bs):
    n, d_in = x.shape
    d_hid = w1.shape[0]

    def body(x_ref, w_ref, d0_ref, d1_ref, o_ref):
        dv = lax.rsqrt(d0_ref[...] + d1_ref[...] + 1.0)
        hm = lax.dot_general(x_ref[...], w_ref[...], (((1,), (1,)), ((), ())),
                             preferred_element_type=jnp.float32)
        o_ref[...] = hm * dv

    return pl.pallas_call(
        body,
        grid=(n // bs,),
        in_specs=[
            pl.BlockSpec((bs, d_in), lambda i: (i, 0)),
            pl.BlockSpec((d_hid, d_in), lambda i: (0, 0)),
            pl.BlockSpec((bs, 1), lambda i: (i, 0)),
            pl.BlockSpec((bs, 1), lambda i: (i, 0)),
        ],
        out_specs=pl.BlockSpec((bs, d_hid), lambda i: (i, 0)),
        out_shape=jax.ShapeDtypeStruct((n, d_hid), jnp.float32),
    )(x, w1, deg0, deg1)


def _tc_layer2(p, h1, deg0, deg1, b1, w2, bs):
    d_hid = h1.shape[1]
    n = p.shape[0] // NC
    d_out = w2.shape[0]
    gn = n // bs

    def body(p0_ref, p1_ref, h_ref, d0_ref, d1_ref, b_ref, w_ref, o_ref):
        dv = lax.rsqrt(d0_ref[...] + d1_ref[...] + 1.0)
        agg = p0_ref[...] + p1_ref[...] - h_ref[...]
        x1 = jnp.maximum(agg * dv + b_ref[...], 0.0)
        h2 = lax.dot_general(x1, w_ref[...], (((1,), (1,)), ((), ())),
                             preferred_element_type=jnp.float32)
        o_ref[...] = h2 * dv

    return pl.pallas_call(
        body,
        grid=(gn,),
        in_specs=[
            pl.BlockSpec((bs, d_hid), lambda i: (i, 0)),
            pl.BlockSpec((bs, d_hid), lambda i: (i + gn, 0)),
            pl.BlockSpec((bs, d_hid), lambda i: (i, 0)),
            pl.BlockSpec((bs, 1), lambda i: (i, 0)),
            pl.BlockSpec((bs, 1), lambda i: (i, 0)),
            pl.BlockSpec((1, d_hid), lambda i: (0, 0)),
            pl.BlockSpec((d_out, d_hid), lambda i: (0, 0)),
        ],
        out_specs=pl.BlockSpec((bs, d_out), lambda i: (i, 0)),
        out_shape=jax.ShapeDtypeStruct((n, d_out), jnp.float32),
    )(p, p, h1, deg0, deg1, b1, w2)


def _tc_layer3(q, h2, deg0, deg1, b2, bs):
    d_out = h2.shape[1]
    n = q.shape[0] // NC
    gn = n // bs

    def body(q0_ref, q1_ref, h_ref, d0_ref, d1_ref, b_ref, o_ref):
        dv = lax.rsqrt(d0_ref[...] + d1_ref[...] + 1.0)
        agg = q0_ref[...] + q1_ref[...] - h_ref[...]
        z = agg * dv + b_ref[...]
        m = jnp.max(z, axis=1, keepdims=True)
        ez = jnp.exp(z - m)
        se = jnp.sum(ez, axis=1, keepdims=True)
        o_ref[...] = z - m - jnp.log(se)

    return pl.pallas_call(
        body,
        grid=(gn,),
        in_specs=[
            pl.BlockSpec((bs, d_out), lambda i: (i, 0)),
            pl.BlockSpec((bs, d_out), lambda i: (i + gn, 0)),
            pl.BlockSpec((bs, d_out), lambda i: (i, 0)),
            pl.BlockSpec((bs, 1), lambda i: (i, 0)),
            pl.BlockSpec((bs, 1), lambda i: (i, 0)),
            pl.BlockSpec((1, d_out), lambda i: (0, 0)),
        ],
        out_specs=pl.BlockSpec((bs, d_out), lambda i: (i, 0)),
        out_shape=jax.ShapeDtypeStruct((n, d_out), jnp.float32),
    )(q, q, h2, deg0, deg1, b2)


def kernel(x, edge_index, W1, b1, W2, b2):
    n, _ = x.shape
    d_hid = W1.shape[0]
    d_out = W2.shape[0]
    e = edge_index.shape[1]
    ei = edge_index.reshape(2 * e)

    per_tile = -(-n // NS)
    per_tile += (-per_tile) % 16
    n_pad = per_tile * NS

    degp = _sc_degree(ei, e, n_pad)
    deg0 = degp[:n].reshape(n, 1)
    deg1 = degp[n_pad:n_pad + n].reshape(n, 1)

    bs = 2000
    h1 = _tc_layer1(x, W1, deg0, deg1, bs)
    p = _sc_aggregate(h1, ei, e)
    h2 = _tc_layer2(p, h1, deg0, deg1, b1.reshape(1, d_hid), W2, bs)
    q = _sc_aggregate(h2, ei, e, tc_tiling=False)
    return _tc_layer3(q, h2, deg0, deg1, b2.reshape(1, d_out), bs)


# speedup vs baseline: 37.3191x; 1.0006x over previous
import jax
import jax.numpy as jnp
from jax import lax
from jax.experimental import pallas as pl
from jax.experimental.pallas import tpu as pltpu
from jax.experimental.pallas import tpu_sc as plsc

NC = 2
NS = 16
NW = NC * NS
CH = 128


def _mesh():
    return plsc.VectorSubcoreMesh(core_axis_name="c", subcore_axis_name="s")


def _sc_degree(ei, e, n_pad):
    per_w = e // NW
    full = per_w // CH
    rem = per_w - full * CH
    per_tile = n_pad // NS

    def body(ei_hbm, out_hbm, didx, ones_v, zb, acc, hs):
        c = lax.axis_index("c")
        s = lax.axis_index("s")
        wid = s * NC + c

        def fill_ones(i, carry):
            ones_v[pl.ds(i * 16, 16)] = jnp.ones((16,), jnp.float32)
            return carry

        lax.fori_loop(0, CH // 16, fill_ones, 0)

        def fill_z(i, carry):
            zb[pl.ds(i * 16, 16)] = jnp.zeros((16,), jnp.float32)
            return carry

        lax.fori_loop(0, per_tile // 16, fill_z, 0)
        pltpu.sync_copy(zb, acc.at[pl.ds(s * per_tile, per_tile)])
        pltpu.sync_copy(ei_hbm.at[pl.ds(e + wid * per_w, per_w)], didx)
        plsc.subcore_barrier()

        def chunk(i, carry):
            pltpu.async_copy(ones_v, acc.at[didx.at[pl.ds(i * CH, CH)]], hs,
                             add=True)
            return carry

        lax.fori_loop(0, full, chunk, 0)

        def drain(i, carry):
            pltpu.make_async_copy(ones_v,
                                  acc.at[didx.at[pl.ds(i * CH, CH)]],
                                  hs).wait()
            return carry

        lax.fori_loop(0, full, drain, 0)
        if rem:
            pltpu.sync_copy(ones_v.at[pl.ds(0, rem)],
                            acc.at[didx.at[pl.ds(full * CH, rem)]], add=True)
        plsc.subcore_barrier()
        pltpu.sync_copy(acc.at[pl.ds(s * per_tile, per_tile)],
                        out_hbm.at[pl.ds(c * n_pad + s * per_tile, per_tile)])

    return pl.kernel(
        body,
        out_type=jax.ShapeDtypeStruct((NC * n_pad,), jnp.float32),
        mesh=_mesh(),
        scratch_types=[
            pltpu.VMEM((per_w,), jnp.int32),
            pltpu.VMEM((CH,), jnp.float32),
            pltpu.VMEM((per_tile,), jnp.float32),
            pltpu.VMEM_SHARED((n_pad,), jnp.float32),
            pltpu.SemaphoreType.DMA,
        ],
    )(ei)


def _sc_aggregate(h, ei, e, tc_tiling=True):
    n, d = h.shape
    per_w = e // NW
    full = per_w // CH
    rem = per_w - full * CH
    ir = 1000
    ni = n // ir
    if d <= 64:
        stages = [(full, 0)]
    else:
        s0 = -(-full // 2)
        s0 += s0 % 2
        stages = [(s0, 0), (full - s0, s0)]
    buf_c = stages[0][0]

    def body(h_hbm, ei_hbm, out_hbm,
             sidx, didx, sidx_r, didx_r, rows, rows_r, acc, gs0, gs1):
        c = lax.axis_index("c")
        s = lax.axis_index("s")
        wid = s * NC + c
        rslice = pl.ds(s * ir, ir)
        base0 = wid * per_w

        def stage_in(sc, off_c):
            off = base0 + off_c * CH
            pltpu.sync_copy(ei_hbm.at[pl.ds(off, sc * CH)],
                            sidx.at[pl.ds(0, sc * CH)])
            pltpu.sync_copy(ei_hbm.at[pl.ds(e + off, sc * CH)],
                            didx.at[pl.ds(0, sc * CH)])
            pltpu.async_copy(h_hbm.at[sidx.at[pl.ds(0, CH)]], rows.at[0], gs0)
            pltpu.async_copy(h_hbm.at[sidx.at[pl.ds(CH, CH)]], rows.at[1],
                             gs1)

        stage_in(*stages[0])
        if rem:
            pltpu.sync_copy(ei_hbm.at[pl.ds(base0 + full * CH, rem)], sidx_r)
            pltpu.sync_copy(ei_hbm.at[pl.ds(e + base0 + full * CH, rem)],
                            didx_r)

        @pl.when(s < ni)
        def _():
            pltpu.sync_copy(h_hbm.at[rslice], acc.at[rslice])

        plsc.subcore_barrier()

        for si, (sc, off_c) in enumerate(stages):
            if si:
                stage_in(sc, off_c)

            def pair(g, carry):
                i0 = 2 * g
                pltpu.make_async_copy(h_hbm.at[sidx.at[pl.ds(i0 * CH, CH)]],
                                      rows.at[0], gs0).wait()
                pltpu.sync_copy(rows.at[0],
                                acc.at[didx.at[pl.ds(i0 * CH, CH)]], add=True)

                @pl.when(i0 + 2 < sc)
                def _():
                    pltpu.async_copy(
                        h_hbm.at[sidx.at[pl.ds((i0 + 2) * CH, CH)]],
                        rows.at[0], gs0)

                pltpu.make_async_copy(
                    h_hbm.at[sidx.at[pl.ds((i0 + 1) * CH, CH)]],
                    rows.at[1], gs1).wait()
                pltpu.sync_copy(rows.at[1],
                                acc.at[didx.at[pl.ds((i0 + 1) * CH, CH)]],
                                add=True)

                @pl.when(i0 + 3 < sc)
                def _():
                    pltpu.async_copy(
                        h_hbm.at[sidx.at[pl.ds((i0 + 3) * CH, CH)]],
                        rows.at[1], gs1)

                return carry

            lax.fori_loop(0, sc // 2, pair, 0)

        if rem:
            pltpu.async_copy(h_hbm.at[sidx_r], rows_r, gs0).wait()
            pltpu.sync_copy(rows_r, acc.at[didx_r], add=True)
        plsc.subcore_barrier()

        @pl.when(s < ni)
        def _():
            pltpu.sync_copy(acc.at[rslice],
                            out_hbm.at[pl.ds(c * n + s * ir, ir)])

    return pl.kernel(
        body,
        out_type=jax.ShapeDtypeStruct((NC * n, d), jnp.float32),
        mesh=_mesh(),
        scratch_types=[
            pltpu.VMEM((buf_c * CH,), jnp.int32),
            pltpu.VMEM((buf_c * CH,), jnp.int32),
            pltpu.VMEM((max(rem, 16),), jnp.int32),
            pltpu.VMEM((max(rem, 16),), jnp.int32),
            pltpu.VMEM((2, CH, d), jnp.float32),
            pltpu.VMEM((max(rem, 16), d), jnp.float32),
            pltpu.VMEM_SHARED((n, d), jnp.float32),
            pltpu.SemaphoreType.DMA,
            pltpu.SemaphoreType.DMA,
        ],
        compiler_params=pltpu.CompilerParams(use_tc_tiling_on_sc=tc_tiling),
    )(h, ei)


def _tc_layer1(x, w1, deg0, deg1, bs):
    n, d_in = x.shape
    d_hid = w1.shape[0]

    def body(x_ref, w_ref, d0_ref, d1_ref, o_ref):
        dv = lax.rsqrt(d0_ref[...] + d1_ref[...] + 1.0)
        hm = lax.dot_general(x_ref[...], w_ref[...], (((1,), (1,)), ((), ())),
                             preferred_element_type=jnp.float32)
        o_ref[...] = hm * dv

    return pl.pallas_call(
        body,
        grid=(n // bs,),
        in_specs=[
            pl.BlockSpec((bs, d_in), lambda i: (i, 0)),
            pl.BlockSpec((d_hid, d_in), lambda i: (0, 0)),
            pl.BlockSpec((bs, 1), lambda i: (i, 0)),
            pl.BlockSpec((bs, 1), lambda i: (i, 0)),
        ],
        out_specs=pl.BlockSpec((bs, d_hid), lambda i: (i, 0)),
        out_shape=jax.ShapeDtypeStruct((n, d_hid), jnp.float32),
    )(x, w1, deg0, deg1)


def _tc_layer2(p, h1, deg0, deg1, b1, w2, bs):
    d_hid = h1.shape[1]
    n = p.shape[0] // NC
    d_out = w2.shape[0]
    gn = n // bs

    def body(p0_ref, p1_ref, h_ref, d0_ref, d1_ref, b_ref, w_ref, o_ref):
        dv = lax.rsqrt(d0_ref[...] + d1_ref[...] + 1.0)
        agg = p0_ref[...] + p1_ref[...] - h_ref[...]
        x1 = jnp.maximum(agg * dv + b_ref[...], 0.0)
        h2 = lax.dot_general(x1, w_ref[...], (((1,), (1,)), ((), ())),
                             preferred_element_type=jnp.float32)
        o_ref[...] = h2 * dv

    return pl.pallas_call(
        body,
        grid=(gn,),
        in_specs=[
            pl.BlockSpec((bs, d_hid), lambda i: (i, 0)),
            pl.BlockSpec((bs, d_hid), lambda i: (i + gn, 0)),
            pl.BlockSpec((bs, d_hid), lambda i: (i, 0)),
            pl.BlockSpec((bs, 1), lambda i: (i, 0)),
            pl.BlockSpec((bs, 1), lambda i: (i, 0)),
            pl.BlockSpec((1, d_hid), lambda i: (0, 0)),
            pl.BlockSpec((d_out, d_hid), lambda i: (0, 0)),
        ],
        out_specs=pl.BlockSpec((bs, d_out), lambda i: (i, 0)),
        out_shape=jax.ShapeDtypeStruct((n, d_out), jnp.float32),
    )(p, p, h1, deg0, deg1, b1, w2)


def _tc_layer3(q, h2, deg0, deg1, b2, bs):
    d_out = h2.shape[1]
    n = q.shape[0] // NC
    gn = n // bs

    def body(q0_ref, q1_ref, h_ref, d0_ref, d1_ref, b_ref, o_ref):
        dv = lax.rsqrt(d0_ref[...] + d1_ref[...] + 1.0)
        agg = q0_ref[...] + q1_ref[...] - h_ref[...]
        z = agg * dv + b_ref[...]
        m = jnp.max(z, axis=1, keepdims=True)
        ez = jnp.exp(z - m)
        se = jnp.sum(ez, axis=1, keepdims=True)
        o_ref[...] = z - m - jnp.log(se)

    return pl.pallas_call(
        body,
        grid=(gn,),
        in_specs=[
            pl.BlockSpec((bs, d_out), lambda i: (i, 0)),
            pl.BlockSpec((bs, d_out), lambda i: (i + gn, 0)),
            pl.BlockSpec((bs, d_out), lambda i: (i, 0)),
            pl.BlockSpec((bs, 1), lambda i: (i, 0)),
            pl.BlockSpec((bs, 1), lambda i: (i, 0)),
            pl.BlockSpec((1, d_out), lambda i: (0, 0)),
        ],
        out_specs=pl.BlockSpec((bs, d_out), lambda i: (i, 0)),
        out_shape=jax.ShapeDtypeStruct((n, d_out), jnp.float32),
    )(q, q, h2, deg0, deg1, b2)


def kernel(x, edge_index, W1, b1, W2, b2):
    n, _ = x.shape
    d_hid = W1.shape[0]
    d_out = W2.shape[0]
    e = edge_index.shape[1]
    ei = edge_index.reshape(2 * e)

    per_tile = -(-n // NS)
    per_tile += (-per_tile) % 16
    n_pad = per_tile * NS

    degp = _sc_degree(ei, e, n_pad)
    deg0 = degp[:n].reshape(n, 1)
    deg1 = degp[n_pad:n_pad + n].reshape(n, 1)

    bs = 2000
    h1 = _tc_layer1(x, W1, deg0, deg1, bs)
    p = _sc_aggregate(h1, ei, e)
    h2 = _tc_layer2(p, h1, deg0, deg1, b1.reshape(1, d_hid), W2, bs)
    q = _sc_aggregate(h2, ei, e, tc_tiling=False)
    return _tc_layer3(q, h2, deg0, deg1, b2.reshape(1, d_out), bs)


# speedup vs baseline: 39.2176x; 1.0509x over previous
import jax
import jax.numpy as jnp
from jax import lax
from jax.experimental import pallas as pl
from jax.experimental.pallas import tpu as pltpu
from jax.experimental.pallas import tpu_sc as plsc

NC = 2
NS = 16
NW = NC * NS
CH = 128


def _mesh():
    return plsc.VectorSubcoreMesh(core_axis_name="c", subcore_axis_name="s")


def _sc_degree(ei, e, n_pad):
    per_w = e // NW
    full = per_w // CH
    rem = per_w - full * CH
    per_tile = n_pad // NS

    def body(ei_hbm, out_hbm, didx, ones_v, zb, acc, hs):
        c = lax.axis_index("c")
        s = lax.axis_index("s")
        wid = s * NC + c

        def fill_ones(i, carry):
            ones_v[pl.ds(i * 16, 16)] = jnp.ones((16,), jnp.float32)
            return carry

        lax.fori_loop(0, CH // 16, fill_ones, 0)

        def fill_z(i, carry):
            zb[pl.ds(i * 16, 16)] = jnp.zeros((16,), jnp.float32)
            return carry

        lax.fori_loop(0, per_tile // 16, fill_z, 0)
        pltpu.sync_copy(zb, acc.at[pl.ds(s * per_tile, per_tile)])
        pltpu.sync_copy(ei_hbm.at[pl.ds(e + wid * per_w, per_w)], didx)
        plsc.subcore_barrier()

        def chunk(i, carry):
            pltpu.async_copy(ones_v, acc.at[didx.at[pl.ds(i * CH, CH)]], hs,
                             add=True)
            return carry

        lax.fori_loop(0, full, chunk, 0)

        def drain(i, carry):
            pltpu.make_async_copy(ones_v,
                                  acc.at[didx.at[pl.ds(i * CH, CH)]],
                                  hs).wait()
            return carry

        lax.fori_loop(0, full, drain, 0)
        if rem:
            pltpu.sync_copy(ones_v.at[pl.ds(0, rem)],
                            acc.at[didx.at[pl.ds(full * CH, rem)]], add=True)
        plsc.subcore_barrier()
        pltpu.sync_copy(acc.at[pl.ds(s * per_tile, per_tile)],
                        out_hbm.at[pl.ds(c * n_pad + s * per_tile, per_tile)])

    return pl.kernel(
        body,
        out_type=jax.ShapeDtypeStruct((NC * n_pad,), jnp.float32),
        mesh=_mesh(),
        scratch_types=[
            pltpu.VMEM((per_w,), jnp.int32),
            pltpu.VMEM((CH,), jnp.float32),
            pltpu.VMEM((per_tile,), jnp.float32),
            pltpu.VMEM_SHARED((n_pad,), jnp.float32),
            pltpu.SemaphoreType.DMA,
        ],
    )(ei)


def _sc_aggregate(h, ei, e, tc_tiling=True):
    n, d = h.shape
    per_w = e // NW
    full = per_w // CH
    rem = per_w - full * CH
    ir = 1000
    ni = n // ir
    if d <= 64:
        stages = [(full, 0)]
        nbuf = 3 if full % 3 == 0 else 2
    else:
        s0 = -(-full // 2)
        s0 += s0 % 2
        stages = [(s0, 0), (full - s0, s0)]
        nbuf = 2
    buf_c = stages[0][0]

    def body(h_hbm, ei_hbm, out_hbm,
             sidx, didx, sidx_r, didx_r, rows, rows_r, acc, *gs):
        c = lax.axis_index("c")
        s = lax.axis_index("s")
        wid = s * NC + c
        rslice = pl.ds(s * ir, ir)
        base0 = wid * per_w

        def stage_in(sc, off_c):
            off = base0 + off_c * CH
            pltpu.sync_copy(ei_hbm.at[pl.ds(off, sc * CH)],
                            sidx.at[pl.ds(0, sc * CH)])
            pltpu.sync_copy(ei_hbm.at[pl.ds(e + off, sc * CH)],
                            didx.at[pl.ds(0, sc * CH)])
            for b in range(nbuf):
                pltpu.async_copy(h_hbm.at[sidx.at[pl.ds(b * CH, CH)]],
                                 rows.at[b], gs[b])

        stage_in(*stages[0])
        if rem:
            pltpu.sync_copy(ei_hbm.at[pl.ds(base0 + full * CH, rem)], sidx_r)
            pltpu.sync_copy(ei_hbm.at[pl.ds(e + base0 + full * CH, rem)],
                            didx_r)

        @pl.when(s < ni)
        def _():
            pltpu.sync_copy(h_hbm.at[rslice], acc.at[rslice])

        plsc.subcore_barrier()

        for si, (sc, off_c) in enumerate(stages):
            if si:
                stage_in(sc, off_c)

            def group(g, carry):
                i0 = nbuf * g
                for b in range(nbuf):
                    i = i0 + b
                    pltpu.make_async_copy(
                        h_hbm.at[sidx.at[pl.ds(i * CH, CH)]],
                        rows.at[b], gs[b]).wait()
                    pltpu.sync_copy(rows.at[b],
                                    acc.at[didx.at[pl.ds(i * CH, CH)]],
                                    add=True)

                    @pl.when(i + nbuf < sc)
                    def _():
                        pltpu.async_copy(
                            h_hbm.at[sidx.at[pl.ds((i + nbuf) * CH, CH)]],
                            rows.at[b], gs[b])

                return carry

            lax.fori_loop(0, sc // nbuf, group, 0)

        if rem:
            pltpu.async_copy(h_hbm.at[sidx_r], rows_r, gs[0]).wait()
            pltpu.sync_copy(rows_r, acc.at[didx_r], add=True)
        plsc.subcore_barrier()

        @pl.when(s < ni)
        def _():
            pltpu.sync_copy(acc.at[rslice],
                            out_hbm.at[pl.ds(c * n + s * ir, ir)])

    return pl.kernel(
        body,
        out_type=jax.ShapeDtypeStruct((NC * n, d), jnp.float32),
        mesh=_mesh(),
        scratch_types=[
            pltpu.VMEM((buf_c * CH,), jnp.int32),
            pltpu.VMEM((buf_c * CH,), jnp.int32),
            pltpu.VMEM((max(rem, 16),), jnp.int32),
            pltpu.VMEM((max(rem, 16),), jnp.int32),
            pltpu.VMEM((nbuf, CH, d), jnp.float32),
            pltpu.VMEM((max(rem, 16), d), jnp.float32),
            pltpu.VMEM_SHARED((n, d), jnp.float32),
        ] + [pltpu.SemaphoreType.DMA] * nbuf,
        compiler_params=pltpu.CompilerParams(use_tc_tiling_on_sc=tc_tiling),
    )(h, ei)


def _tc_layer1(x, w1, deg0, deg1, bs):
    n, d_in = x.shape
    d_hid = w1.shape[0]

    def body(x_ref, w_ref, d0_ref, d1_ref, o_ref):
        dv = lax.rsqrt(d0_ref[...] + d1_ref[...] + 1.0)
        hm = lax.dot_general(x_ref[...], w_ref[...], (((1,), (1,)), ((), ())),
                             preferred_element_type=jnp.float32)
        o_ref[...] = hm * dv

    return pl.pallas_call(
        body,
        grid=(n // bs,),
        in_specs=[
            pl.BlockSpec((bs, d_in), lambda i: (i, 0)),
            pl.BlockSpec((d_hid, d_in), lambda i: (0, 0)),
            pl.BlockSpec((bs, 1), lambda i: (i, 0)),
            pl.BlockSpec((bs, 1), lambda i: (i, 0)),
        ],
        out_specs=pl.BlockSpec((bs, d_hid), lambda i: (i, 0)),
        out_shape=jax.ShapeDtypeStruct((n, d_hid), jnp.float32),
    )(x, w1, deg0, deg1)


def _tc_layer2(p, h1, deg0, deg1, b1, w2, bs):
    d_hid = h1.shape[1]
    n = p.shape[0] // NC
    d_out = w2.shape[0]
    gn = n // bs

    def body(p0_ref, p1_ref, h_ref, d0_ref, d1_ref, b_ref, w_ref, o_ref):
        dv = lax.rsqrt(d0_ref[...] + d1_ref[...] + 1.0)
        agg = p0_ref[...] + p1_ref[...] - h_ref[...]
        x1 = jnp.maximum(agg * dv + b_ref[...], 0.0)
        h2 = lax.dot_general(x1, w_ref[...], (((1,), (1,)), ((), ())),
                             preferred_element_type=jnp.float32)
        o_ref[...] = h2 * dv

    return pl.pallas_call(
        body,
        grid=(gn,),
        in_specs=[
            pl.BlockSpec((bs, d_hid), lambda i: (i, 0)),
            pl.BlockSpec((bs, d_hid), lambda i: (i + gn, 0)),
            pl.BlockSpec((bs, d_hid), lambda i: (i, 0)),
            pl.BlockSpec((bs, 1), lambda i: (i, 0)),
            pl.BlockSpec((bs, 1), lambda i: (i, 0)),
            pl.BlockSpec((1, d_hid), lambda i: (0, 0)),
            pl.BlockSpec((d_out, d_hid), lambda i: (0, 0)),
        ],
        out_specs=pl.BlockSpec((bs, d_out), lambda i: (i, 0)),
        out_shape=jax.ShapeDtypeStruct((n, d_out), jnp.float32),
    )(p, p, h1, deg0, deg1, b1, w2)


def _tc_layer3(q, h2, deg0, deg1, b2, bs):
    d_out = h2.shape[1]
    n = q.shape[0] // NC
    gn = n // bs

    def body(q0_ref, q1_ref, h_ref, d0_ref, d1_ref, b_ref, o_ref):
        dv = lax.rsqrt(d0_ref[...] + d1_ref[...] + 1.0)
        agg = q0_ref[...] + q1_ref[...] - h_ref[...]
        z = agg * dv + b_ref[...]
        m = jnp.max(z, axis=1, keepdims=True)
        ez = jnp.exp(z - m)
        se = jnp.sum(ez, axis=1, keepdims=True)
        o_ref[...] = z - m - jnp.log(se)

    return pl.pallas_call(
        body,
        grid=(gn,),
        in_specs=[
            pl.BlockSpec((bs, d_out), lambda i: (i, 0)),
            pl.BlockSpec((bs, d_out), lambda i: (i + gn, 0)),
            pl.BlockSpec((bs, d_out), lambda i: (i, 0)),
            pl.BlockSpec((bs, 1), lambda i: (i, 0)),
            pl.BlockSpec((bs, 1), lambda i: (i, 0)),
            pl.BlockSpec((1, d_out), lambda i: (0, 0)),
        ],
        out_specs=pl.BlockSpec((bs, d_out), lambda i: (i, 0)),
        out_shape=jax.ShapeDtypeStruct((n, d_out), jnp.float32),
    )(q, q, h2, deg0, deg1, b2)


def kernel(x, edge_index, W1, b1, W2, b2):
    n, _ = x.shape
    d_hid = W1.shape[0]
    d_out = W2.shape[0]
    e = edge_index.shape[1]
    ei = edge_index.reshape(2 * e)

    per_tile = -(-n // NS)
    per_tile += (-per_tile) % 16
    n_pad = per_tile * NS

    degp = _sc_degree(ei, e, n_pad)
    deg0 = degp[:n].reshape(n, 1)
    deg1 = degp[n_pad:n_pad + n].reshape(n, 1)

    bs = 2000
    h1 = _tc_layer1(x, W1, deg0, deg1, bs)
    p = _sc_aggregate(h1, ei, e)
    h2 = _tc_layer2(p, h1, deg0, deg1, b1.reshape(1, d_hid), W2, bs)
    q = _sc_aggregate(h2, ei, e, tc_tiling=False)
    return _tc_layer3(q, h2, deg0, deg1, b2.reshape(1, d_out), bs)
